# baseline (speedup 1.0000x reference)
import functools

import jax
import jax.numpy as jnp
from jax import lax
from jax.experimental import pallas as pl
from jax.experimental.pallas import tpu as pltpu
from jax.experimental.pallas import tpu_sc as plsc

VOCAB = 1000000
EMBED = 64
BATCH = 16384
NUM_NEG = 20

NC, NS, L = 2, 16, 16
NW = NC * NS
B_PER_W = BATCH // NW
PHYS = 2 * EMBED
CHUNK = 32
NSTEPS = B_PER_W // CHUNK
NEG_ROWS = CHUNK * NUM_NEG
NSPLIT = NEG_ROWS // 128
NEG_PER_W = B_PER_W * NUM_NEG


def _iota16():
    return lax.iota(jnp.int32, L)


def _sc_scores(center, context, neg_flat, wc2, wx2):
    mesh = plsc.VectorSubcoreMesh(core_axis_name="c", subcore_axis_name="s")

    @functools.partial(
        pl.kernel,
        out_type=(
            jax.ShapeDtypeStruct((BATCH,), jnp.float32),
            jax.ShapeDtypeStruct((BATCH * NUM_NEG,), jnp.float32),
        ),
        mesh=mesh,
        scratch_types=[
            pltpu.VMEM((B_PER_W,), jnp.int32),
            pltpu.VMEM((B_PER_W,), jnp.int32),
            pltpu.VMEM((NEG_PER_W,), jnp.int32),
            pltpu.VMEM((B_PER_W,), jnp.int32),
            pltpu.VMEM((B_PER_W,), jnp.int32),
            pltpu.VMEM((NEG_PER_W,), jnp.int32),
            pltpu.VMEM((CHUNK, PHYS), jnp.float32),
            pltpu.VMEM((CHUNK, PHYS), jnp.float32),
            pltpu.VMEM((NEG_ROWS, PHYS), jnp.float32),
            pltpu.VMEM((B_PER_W,), jnp.float32),
            pltpu.VMEM((NEG_PER_W,), jnp.float32),
            pltpu.SemaphoreType.DMA,
        ],
        compiler_params=pltpu.CompilerParams(
            needs_layout_passes=False, use_tc_tiling_on_sc=True),
    )
    def scores_kernel(center_h, context_h, neg_h, wc_h, wx_h,
                      pos_h, neg_out_h,
                      raw_c, raw_x, raw_n, row_c, row_x, row_n,
                      rows_c, rows_x, rows_n, pos_v, neg_v, sem):
        wid = lax.axis_index("s") * NC + lax.axis_index("c")
        base = wid * B_PER_W

        pltpu.sync_copy(center_h.at[pl.ds(base, B_PER_W)], raw_c)
        pltpu.sync_copy(context_h.at[pl.ds(base, B_PER_W)], raw_x)
        pltpu.sync_copy(neg_h.at[pl.ds(base * NUM_NEG, NEG_PER_W)], raw_n)

        def shift_body(i, _, src, dst):
            v16 = i * L + _iota16()
            x = plsc.load_gather(src, [v16])
            plsc.store_scatter(dst, [v16], x >> 1)
            return 0

        lax.fori_loop(0, B_PER_W // L,
                      functools.partial(shift_body, src=raw_c, dst=row_c), 0)
        lax.fori_loop(0, B_PER_W // L,
                      functools.partial(shift_body, src=raw_x, dst=row_x), 0)
        lax.fori_loop(0, NEG_PER_W // L,
                      functools.partial(shift_body, src=raw_n, dst=row_n), 0)

        def step_body(step, _):
            cb = step * CHUNK
            nb = step * NEG_ROWS
            descs = [
                pltpu.async_copy(wc_h.at[row_c.at[pl.ds(cb, CHUNK)]],
                                 rows_c, sem),
                pltpu.async_copy(wx_h.at[row_x.at[pl.ds(cb, CHUNK)]],
                                 rows_x, sem),
            ]
            for j in range(NSPLIT):
                descs.append(pltpu.async_copy(
                    wx_h.at[row_n.at[pl.ds(nb + j * 128, 128)]],
                    rows_n.at[pl.ds(j * 128, 128)], sem))
            for d in descs:
                d.wait()

            for g in range(CHUNK // L):
                loc16 = _iota16() + g * L
                tb = loc16 + cb
                tb20 = tb * NUM_NEG
                rowb = loc16 * NUM_NEG
                colc = (plsc.load_gather(raw_c, [tb]) & 1) << 6
                colx = (plsc.load_gather(raw_x, [tb]) & 1) << 6
                coln = [(plsc.load_gather(raw_n, [tb20 + k]) & 1) << 6
                        for k in range(NUM_NEG)]

                def dim_body(dd, accs, loc16=loc16, rowb=rowb,
                             colc=colc, colx=colx, coln=coln):
                    v = plsc.load_gather(rows_c, [loc16, colc + dd])
                    up = plsc.load_gather(rows_x, [loc16, colx + dd])
                    new = [accs[0] + v * up]
                    for k in range(NUM_NEG):
                        un = plsc.load_gather(
                            rows_n, [rowb + k, coln[k] + dd])
                        new.append(accs[k + 1] + v * un)
                    return tuple(new)

                accs = lax.fori_loop(
                    0, EMBED, dim_body,
                    tuple(jnp.zeros((L,), jnp.float32)
                          for _ in range(NUM_NEG + 1)))
                plsc.store_scatter(pos_v, [tb], accs[0])
                for k in range(NUM_NEG):
                    plsc.store_scatter(neg_v, [tb20 + k], accs[k + 1])
            return 0

        lax.fori_loop(0, NSTEPS, step_body, 0)
        pltpu.sync_copy(pos_v, pos_h.at[pl.ds(base, B_PER_W)])
        pltpu.sync_copy(neg_v, neg_out_h.at[pl.ds(base * NUM_NEG, NEG_PER_W)])

    return scores_kernel(center, context, neg_flat, wc2, wx2)


def _loss_kernel(pos_ref, neg_ref, out_ref):
    def log_sigmoid(x):
        return jnp.minimum(x, 0.0) - jnp.log1p(jnp.exp(-jnp.abs(x)))

    total = (jnp.sum(log_sigmoid(pos_ref[...]))
             + jnp.sum(log_sigmoid(-neg_ref[...])))
    out_ref[0, 0] = -total / BATCH


def kernel(center, context, negatives, W_center, W_context):
    center = center.astype(jnp.int32)
    context = context.astype(jnp.int32)
    neg_flat = negatives.astype(jnp.int32).reshape(BATCH * NUM_NEG)
    wc2 = W_center.reshape(VOCAB // 2, PHYS)
    wx2 = W_context.reshape(VOCAB // 2, PHYS)
    pos, neg = _sc_scores(center, context, neg_flat, wc2, wx2)
    loss = pl.pallas_call(
        _loss_kernel,
        out_shape=jax.ShapeDtypeStruct((1, 1), jnp.float32),
        in_specs=[
            pl.BlockSpec(memory_space=pltpu.VMEM),
            pl.BlockSpec(memory_space=pltpu.VMEM),
        ],
        out_specs=pl.BlockSpec(memory_space=pltpu.SMEM),
    )(pos.reshape(BATCH // 128, 128), neg.reshape(BATCH * NUM_NEG // 128, 128))
    return loss[0, 0]


# speedup vs baseline: 1.5464x; 1.5464x over previous
import functools

import jax
import jax.numpy as jnp
from jax import lax
from jax.experimental import pallas as pl
from jax.experimental.pallas import tpu as pltpu
from jax.experimental.pallas import tpu_sc as plsc

VOCAB = 1000000
EMBED = 64
BATCH = 16384
NUM_NEG = 20

NC, NS, L = 2, 16, 16
NW = NC * NS
B_PER_W = BATCH // NW
PHYS = 2 * EMBED
CHUNK = 32
NSTEPS = B_PER_W // CHUNK
NEG_ROWS = CHUNK * NUM_NEG
NSPLIT = NEG_ROWS // 128
NEG_PER_W = B_PER_W * NUM_NEG

TBLK = 2048
NTBLK = 245
SPLIT = NTBLK * TBLK
NRBLK = (VOCAB - SPLIT + TBLK - 1) // TBLK + NTBLK


def _transpose_pack(wt_c, wt_x):
    def body(c1_ref, c2_ref, x1_ref, x2_ref, oc_ref, ox_ref):
        eye = (lax.broadcasted_iota(jnp.int32, (EMBED, EMBED), 0)
               == lax.broadcasted_iota(jnp.int32, (EMBED, EMBED), 1)
               ).astype(jnp.float32)

        def tr(ref):
            return lax.dot_general(ref[...], eye, (((0,), (0,)), ((), ())),
                                   preferred_element_type=jnp.float32)

        oc_ref[...] = jnp.concatenate([tr(c1_ref), tr(c2_ref)], axis=1)
        ox_ref[...] = jnp.concatenate([tr(x1_ref), tr(x2_ref)], axis=1)

    left = lambda b: (0, b)
    right = lambda b: (0, jnp.minimum(b + NTBLK, NRBLK - 1))
    in_spec = [pl.BlockSpec((EMBED, TBLK), m) for m in (left, right)] * 2
    out_spec = pl.BlockSpec((TBLK, PHYS), lambda b: (b, 0))
    return pl.pallas_call(
        body,
        grid=(NTBLK,),
        in_specs=in_spec,
        out_specs=[out_spec, out_spec],
        out_shape=[jax.ShapeDtypeStruct((SPLIT, PHYS), jnp.float32)] * 2,
    )(wt_c, wt_c, wt_x, wt_x)


def _sc_scores(center, context, neg_flat, wc2, wx2):
    mesh = plsc.VectorSubcoreMesh(core_axis_name="c", subcore_axis_name="s")

    @functools.partial(
        pl.kernel,
        out_type=(
            jax.ShapeDtypeStruct((BATCH,), jnp.float32),
            jax.ShapeDtypeStruct((BATCH * NUM_NEG,), jnp.float32),
        ),
        mesh=mesh,
        scratch_types=[
            pltpu.VMEM((B_PER_W,), jnp.int32),
            pltpu.VMEM((B_PER_W,), jnp.int32),
            pltpu.VMEM((NEG_PER_W,), jnp.int32),
            pltpu.VMEM((B_PER_W,), jnp.int32),
            pltpu.VMEM((B_PER_W,), jnp.int32),
            pltpu.VMEM((NEG_PER_W,), jnp.int32),
            pltpu.VMEM((CHUNK, PHYS), jnp.float32),
            pltpu.VMEM((CHUNK, PHYS), jnp.float32),
            pltpu.VMEM((NEG_ROWS, PHYS), jnp.float32),
            pltpu.VMEM((B_PER_W,), jnp.float32),
            pltpu.VMEM((NEG_PER_W,), jnp.float32),
            pltpu.SemaphoreType.DMA,
        ],
        compiler_params=pltpu.CompilerParams(
            needs_layout_passes=False, use_tc_tiling_on_sc=True),
    )
    def scores_kernel(center_h, context_h, neg_h, wc_h, wx_h,
                      pos_h, neg_out_h,
                      raw_c, raw_x, raw_n, row_c, row_x, row_n,
                      rows_c, rows_x, rows_n, pos_v, neg_v, sem):
        wid = lax.axis_index("s") * NC + lax.axis_index("c")
        base = wid * B_PER_W

        pltpu.sync_copy(center_h.at[pl.ds(base, B_PER_W)], raw_c)
        pltpu.sync_copy(context_h.at[pl.ds(base, B_PER_W)], raw_x)
        pltpu.sync_copy(neg_h.at[pl.ds(base * NUM_NEG, NEG_PER_W)], raw_n)

        def shift_body(i, _, src, dst):
            v16 = i * L + _iota16()
            x = plsc.load_gather(src, [v16])
            row = jnp.where(x >= SPLIT, x - SPLIT, x)
            plsc.store_scatter(dst, [v16], row)
            return 0

        lax.fori_loop(0, B_PER_W // L,
                      functools.partial(shift_body, src=raw_c, dst=row_c), 0)
        lax.fori_loop(0, B_PER_W // L,
                      functools.partial(shift_body, src=raw_x, dst=row_x), 0)
        lax.fori_loop(0, NEG_PER_W // L,
                      functools.partial(shift_body, src=raw_n, dst=row_n), 0)

        def colbase(raw_vec):
            return jnp.where(raw_vec >= SPLIT, EMBED, 0)

        def step_body(step, _):
            cb = step * CHUNK
            nb = step * NEG_ROWS
            descs = [
                pltpu.async_copy(wc_h.at[row_c.at[pl.ds(cb, CHUNK)]],
                                 rows_c, sem),
                pltpu.async_copy(wx_h.at[row_x.at[pl.ds(cb, CHUNK)]],
                                 rows_x, sem),
            ]
            for j in range(NSPLIT):
                descs.append(pltpu.async_copy(
                    wx_h.at[row_n.at[pl.ds(nb + j * 128, 128)]],
                    rows_n.at[pl.ds(j * 128, 128)], sem))
            for d in descs:
                d.wait()

            for g in range(CHUNK // L):
                loc16 = _iota16() + g * L
                tb = loc16 + cb
                tb20 = tb * NUM_NEG
                rowb = loc16 * NUM_NEG
                colc = colbase(plsc.load_gather(raw_c, [tb]))
                colx = colbase(plsc.load_gather(raw_x, [tb]))
                coln = [colbase(plsc.load_gather(raw_n, [tb20 + k]))
                        for k in range(NUM_NEG)]

                def dim_body(dd, accs, loc16=loc16, rowb=rowb,
                             colc=colc, colx=colx, coln=coln):
                    v = plsc.load_gather(rows_c, [loc16, colc + dd])
                    up = plsc.load_gather(rows_x, [loc16, colx + dd])
                    new = [accs[0] + v * up]
                    for k in range(NUM_NEG):
                        un = plsc.load_gather(
                            rows_n, [rowb + k, coln[k] + dd])
                        new.append(accs[k + 1] + v * un)
                    return tuple(new)

                accs = lax.fori_loop(
                    0, EMBED, dim_body,
                    tuple(jnp.zeros((L,), jnp.float32)
                          for _ in range(NUM_NEG + 1)))
                plsc.store_scatter(pos_v, [tb], accs[0])
                for k in range(NUM_NEG):
                    plsc.store_scatter(neg_v, [tb20 + k], accs[k + 1])
            return 0

        lax.fori_loop(0, NSTEPS, step_body, 0)
        pltpu.sync_copy(pos_v, pos_h.at[pl.ds(base, B_PER_W)])
        pltpu.sync_copy(neg_v, neg_out_h.at[pl.ds(base * NUM_NEG, NEG_PER_W)])

    return scores_kernel(center, context, neg_flat, wc2, wx2)


def _iota16():
    return lax.iota(jnp.int32, L)


def _loss_kernel(pos_ref, neg_ref, out_ref):
    def log_sigmoid(x):
        return jnp.minimum(x, 0.0) - jnp.log1p(jnp.exp(-jnp.abs(x)))

    total = (jnp.sum(log_sigmoid(pos_ref[...]))
             + jnp.sum(log_sigmoid(-neg_ref[...])))
    out_ref[0, 0] = -total / BATCH


def kernel(center, context, negatives, W_center, W_context):
    center = center.astype(jnp.int32)
    context = context.astype(jnp.int32)
    neg_flat = negatives.astype(jnp.int32).reshape(BATCH * NUM_NEG)
    wc2, wx2 = _transpose_pack(W_center.T, W_context.T)
    pos, neg = _sc_scores(center, context, neg_flat, wc2, wx2)
    loss = pl.pallas_call(
        _loss_kernel,
        out_shape=jax.ShapeDtypeStruct((1, 1), jnp.float32),
        in_specs=[
            pl.BlockSpec(memory_space=pltpu.VMEM),
            pl.BlockSpec(memory_space=pltpu.VMEM),
        ],
        out_specs=pl.BlockSpec(memory_space=pltpu.SMEM),
    )(pos.reshape(BATCH // 128, 128), neg.reshape(BATCH * NUM_NEG // 128, 128))
    return loss[0, 0]


# speedup vs baseline: 1.5555x; 1.0059x over previous
import functools

import jax
import jax.numpy as jnp
from jax import lax
from jax.experimental import pallas as pl
from jax.experimental.pallas import tpu as pltpu
from jax.experimental.pallas import tpu_sc as plsc

VOCAB = 1000000
EMBED = 64
BATCH = 16384
NUM_NEG = 20

NC, NS, L = 2, 16, 16
NW = NC * NS
B_PER_W = BATCH // NW
PHYS = 2 * EMBED
CHUNK = 32
NSTEPS = B_PER_W // CHUNK
NEG_ROWS = CHUNK * NUM_NEG
NSPLIT = NEG_ROWS // 128
NEG_PER_W = B_PER_W * NUM_NEG

TBLK = 2048
NTBLK = 245
SPLIT = NTBLK * TBLK
NRBLK = (VOCAB - SPLIT + TBLK - 1) // TBLK + NTBLK


def _transpose_pack(wt_c, wt_x):
    def body(c1_ref, c2_ref, x1_ref, x2_ref, oc_ref, ox_ref):
        eye = (lax.broadcasted_iota(jnp.int32, (EMBED, EMBED), 0)
               == lax.broadcasted_iota(jnp.int32, (EMBED, EMBED), 1)
               ).astype(jnp.float32)

        def tr(ref):
            return lax.dot_general(ref[...], eye, (((0,), (0,)), ((), ())),
                                   preferred_element_type=jnp.float32)

        oc_ref[...] = jnp.concatenate([tr(c1_ref), tr(c2_ref)], axis=1)
        ox_ref[...] = jnp.concatenate([tr(x1_ref), tr(x2_ref)], axis=1)

    left = lambda b: (0, b)
    right = lambda b: (0, jnp.minimum(b + NTBLK, NRBLK - 1))
    in_spec = [pl.BlockSpec((EMBED, TBLK), m) for m in (left, right)] * 2
    out_spec = pl.BlockSpec((TBLK, PHYS), lambda b: (b, 0))
    return pl.pallas_call(
        body,
        grid=(NTBLK,),
        in_specs=in_spec,
        out_specs=[out_spec, out_spec],
        out_shape=[jax.ShapeDtypeStruct((SPLIT, PHYS), jnp.float32)] * 2,
    )(wt_c, wt_c, wt_x, wt_x)


def _sc_scores(center, context, neg_flat, wc2, wx2):
    mesh = plsc.VectorSubcoreMesh(core_axis_name="c", subcore_axis_name="s")

    @functools.partial(
        pl.kernel,
        out_type=(
            jax.ShapeDtypeStruct((BATCH,), jnp.float32),
            jax.ShapeDtypeStruct((BATCH * NUM_NEG,), jnp.float32),
        ),
        mesh=mesh,
        scratch_types=[
            pltpu.VMEM((B_PER_W,), jnp.int32),
            pltpu.VMEM((B_PER_W,), jnp.int32),
            pltpu.VMEM((NEG_PER_W,), jnp.int32),
            pltpu.VMEM((B_PER_W,), jnp.int32),
            pltpu.VMEM((B_PER_W,), jnp.int32),
            pltpu.VMEM((NEG_PER_W,), jnp.int32),
            pltpu.VMEM((CHUNK, PHYS), jnp.float32),
            pltpu.VMEM((CHUNK, PHYS), jnp.float32),
            pltpu.VMEM((NEG_ROWS, PHYS), jnp.float32),
            pltpu.VMEM((B_PER_W,), jnp.float32),
            pltpu.VMEM((NEG_PER_W,), jnp.float32),
            pltpu.SemaphoreType.DMA,
        ],
        compiler_params=pltpu.CompilerParams(
            needs_layout_passes=False, use_tc_tiling_on_sc=True),
    )
    def scores_kernel(center_h, context_h, neg_h, wc_h, wx_h,
                      pos_h, neg_out_h,
                      raw_c, raw_x, raw_n, row_c, row_x, row_n,
                      rows_c, rows_x, rows_n, pos_v, neg_v, sem):
        wid = lax.axis_index("s") * NC + lax.axis_index("c")
        base = wid * B_PER_W

        pltpu.sync_copy(center_h.at[pl.ds(base, B_PER_W)], raw_c)
        pltpu.sync_copy(context_h.at[pl.ds(base, B_PER_W)], raw_x)
        pltpu.sync_copy(neg_h.at[pl.ds(base * NUM_NEG, NEG_PER_W)], raw_n)

        def shift_body(i, _, src, dst):
            v16 = i * L + _iota16()
            x = plsc.load_gather(src, [v16])
            row = jnp.where(x >= SPLIT, x - SPLIT, x)
            plsc.store_scatter(dst, [v16], row)
            return 0

        lax.fori_loop(0, B_PER_W // L,
                      functools.partial(shift_body, src=raw_c, dst=row_c), 0)
        lax.fori_loop(0, B_PER_W // L,
                      functools.partial(shift_body, src=raw_x, dst=row_x), 0)
        lax.fori_loop(0, NEG_PER_W // L,
                      functools.partial(shift_body, src=raw_n, dst=row_n), 0)

        def colbase(raw_vec):
            return jnp.where(raw_vec >= SPLIT, EMBED, 0)

        def step_body(step, _):
            cb = step * CHUNK
            nb = step * NEG_ROWS
            descs = [
                pltpu.async_copy(wc_h.at[row_c.at[pl.ds(cb, CHUNK)]],
                                 rows_c, sem),
                pltpu.async_copy(wx_h.at[row_x.at[pl.ds(cb, CHUNK)]],
                                 rows_x, sem),
            ]
            for j in range(NSPLIT):
                descs.append(pltpu.async_copy(
                    wx_h.at[row_n.at[pl.ds(nb + j * 128, 128)]],
                    rows_n.at[pl.ds(j * 128, 128)], sem))
            for d in descs:
                d.wait()

            for g in range(CHUNK // L):
                loc16 = _iota16() + g * L
                tb = loc16 + cb
                tb20 = tb * NUM_NEG
                rowb = loc16 * NUM_NEG
                colc = colbase(plsc.load_gather(raw_c, [tb]))
                KH = NUM_NEG // 2

                colx = colbase(plsc.load_gather(raw_x, [tb]))
                coln = [colbase(plsc.load_gather(raw_n, [tb20 + k]))
                        for k in range(KH)]
                rowk = [rowb + k for k in range(KH)]

                def body_a(dd, accs, loc16=loc16, colc=colc, colx=colx,
                           coln=coln, rowk=rowk):
                    v = plsc.load_gather(rows_c, [loc16, colc + dd])
                    up = plsc.load_gather(rows_x, [loc16, colx + dd])
                    new = [accs[0] + v * up]
                    for k in range(KH):
                        un = plsc.load_gather(
                            rows_n, [rowk[k], coln[k] + dd])
                        new.append(accs[k + 1] + v * un)
                    return tuple(new)

                accs = lax.fori_loop(
                    0, EMBED, body_a,
                    tuple(jnp.zeros((L,), jnp.float32)
                          for _ in range(KH + 1)))
                plsc.store_scatter(pos_v, [tb], accs[0])
                for k in range(KH):
                    plsc.store_scatter(neg_v, [tb20 + k], accs[k + 1])

                coln = [colbase(plsc.load_gather(raw_n, [tb20 + k]))
                        for k in range(KH, NUM_NEG)]
                rowk = [rowb + k for k in range(KH, NUM_NEG)]

                def body_b(dd, accs, loc16=loc16, colc=colc,
                           coln=coln, rowk=rowk):
                    v = plsc.load_gather(rows_c, [loc16, colc + dd])
                    new = []
                    for k in range(KH):
                        un = plsc.load_gather(
                            rows_n, [rowk[k], coln[k] + dd])
                        new.append(accs[k] + v * un)
                    return tuple(new)

                accs = lax.fori_loop(
                    0, EMBED, body_b,
                    tuple(jnp.zeros((L,), jnp.float32)
                          for _ in range(KH)))
                for k in range(KH):
                    plsc.store_scatter(neg_v, [tb20 + KH + k], accs[k])
            return 0

        lax.fori_loop(0, NSTEPS, step_body, 0)
        pltpu.sync_copy(pos_v, pos_h.at[pl.ds(base, B_PER_W)])
        pltpu.sync_copy(neg_v, neg_out_h.at[pl.ds(base * NUM_NEG, NEG_PER_W)])

    return scores_kernel(center, context, neg_flat, wc2, wx2)


def _iota16():
    return lax.iota(jnp.int32, L)


def _loss_kernel(pos_ref, neg_ref, out_ref):
    def log_sigmoid(x):
        return jnp.minimum(x, 0.0) - jnp.log1p(jnp.exp(-jnp.abs(x)))

    total = (jnp.sum(log_sigmoid(pos_ref[...]))
             + jnp.sum(log_sigmoid(-neg_ref[...])))
    out_ref[0, 0] = -total / BATCH


def kernel(center, context, negatives, W_center, W_context):
    center = center.astype(jnp.int32)
    context = context.astype(jnp.int32)
    neg_flat = negatives.astype(jnp.int32).reshape(BATCH * NUM_NEG)
    wc2, wx2 = _transpose_pack(W_center.T, W_context.T)
    pos, neg = _sc_scores(center, context, neg_flat, wc2, wx2)
    loss = pl.pallas_call(
        _loss_kernel,
        out_shape=jax.ShapeDtypeStruct((1, 1), jnp.float32),
        in_specs=[
            pl.BlockSpec(memory_space=pltpu.VMEM),
            pl.BlockSpec(memory_space=pltpu.VMEM),
        ],
        out_specs=pl.BlockSpec(memory_space=pltpu.SMEM),
    )(pos.reshape(BATCH // 128, 128), neg.reshape(BATCH * NUM_NEG // 128, 128))
    return loss[0, 0]


# speedup vs baseline: 1.6627x; 1.0689x over previous
import functools

import jax
import jax.numpy as jnp
from jax import lax
from jax.experimental import pallas as pl
from jax.experimental.pallas import tpu as pltpu
from jax.experimental.pallas import tpu_sc as plsc

VOCAB = 1000000
EMBED = 64
BATCH = 16384
NUM_NEG = 20

NC, NS, L = 2, 16, 16
NW = NC * NS
B_PER_W = BATCH // NW
PHYS = 2 * EMBED
CHUNK = 16
NSTEPS = B_PER_W // CHUNK
NEG_ROWS = CHUNK * NUM_NEG
NSEG = (128, 128, 64)
NEG_PER_W = B_PER_W * NUM_NEG

TBLK = 2048
NTBLK = 245
SPLIT = NTBLK * TBLK
NRBLK = (VOCAB - SPLIT + TBLK - 1) // TBLK + NTBLK


def _transpose_pack(wt_c, wt_x):
    def body(c1_ref, c2_ref, x1_ref, x2_ref, oc_ref, ox_ref):
        eye = (lax.broadcasted_iota(jnp.int32, (EMBED, EMBED), 0)
               == lax.broadcasted_iota(jnp.int32, (EMBED, EMBED), 1)
               ).astype(jnp.float32)

        def tr(ref):
            return lax.dot_general(ref[...], eye, (((0,), (0,)), ((), ())),
                                   preferred_element_type=jnp.float32)

        oc_ref[...] = jnp.concatenate([tr(c1_ref), tr(c2_ref)], axis=1)
        ox_ref[...] = jnp.concatenate([tr(x1_ref), tr(x2_ref)], axis=1)

    left = lambda b: (0, b)
    right = lambda b: (0, jnp.minimum(b + NTBLK, NRBLK - 1))
    in_spec = [pl.BlockSpec((EMBED, TBLK), m) for m in (left, right)] * 2
    out_spec = pl.BlockSpec((TBLK, PHYS), lambda b: (b, 0))
    return pl.pallas_call(
        body,
        grid=(NTBLK,),
        in_specs=in_spec,
        out_specs=[out_spec, out_spec],
        out_shape=[jax.ShapeDtypeStruct((SPLIT, PHYS), jnp.float32)] * 2,
    )(wt_c, wt_c, wt_x, wt_x)


def _sc_scores(center, context, neg_flat, wc2, wx2):
    mesh = plsc.VectorSubcoreMesh(core_axis_name="c", subcore_axis_name="s")

    @functools.partial(
        pl.kernel,
        out_type=(
            jax.ShapeDtypeStruct((BATCH,), jnp.float32),
            jax.ShapeDtypeStruct((BATCH * NUM_NEG,), jnp.float32),
        ),
        mesh=mesh,
        scratch_types=[
            pltpu.VMEM((B_PER_W,), jnp.int32),
            pltpu.VMEM((B_PER_W,), jnp.int32),
            pltpu.VMEM((NEG_PER_W,), jnp.int32),
            pltpu.VMEM((B_PER_W,), jnp.int32),
            pltpu.VMEM((B_PER_W,), jnp.int32),
            pltpu.VMEM((NEG_PER_W,), jnp.int32),
            pltpu.VMEM((CHUNK, PHYS), jnp.float32),
            pltpu.VMEM((CHUNK, PHYS), jnp.float32),
            pltpu.VMEM((NEG_ROWS, PHYS), jnp.float32),
            pltpu.VMEM((CHUNK, PHYS), jnp.float32),
            pltpu.VMEM((CHUNK, PHYS), jnp.float32),
            pltpu.VMEM((NEG_ROWS, PHYS), jnp.float32),
            pltpu.VMEM((B_PER_W,), jnp.float32),
            pltpu.VMEM((NEG_PER_W,), jnp.float32),
            pltpu.SemaphoreType.DMA,
            pltpu.SemaphoreType.DMA,
        ],
        compiler_params=pltpu.CompilerParams(
            needs_layout_passes=False, use_tc_tiling_on_sc=True),
    )
    def scores_kernel(center_h, context_h, neg_h, wc_h, wx_h,
                      pos_h, neg_out_h,
                      raw_c, raw_x, raw_n, row_c, row_x, row_n,
                      rows_cA, rows_xA, rows_nA, rows_cB, rows_xB, rows_nB,
                      pos_v, neg_v, semA, semB):
        wid = lax.axis_index("s") * NC + lax.axis_index("c")
        base = wid * B_PER_W

        pltpu.sync_copy(center_h.at[pl.ds(base, B_PER_W)], raw_c)
        pltpu.sync_copy(context_h.at[pl.ds(base, B_PER_W)], raw_x)
        pltpu.sync_copy(neg_h.at[pl.ds(base * NUM_NEG, NEG_PER_W)], raw_n)

        def shift_body(i, _, src, dst):
            v16 = i * L + _iota16()
            x = plsc.load_gather(src, [v16])
            row = jnp.where(x >= SPLIT, x - SPLIT, x)
            plsc.store_scatter(dst, [v16], row)
            return 0

        lax.fori_loop(0, B_PER_W // L,
                      functools.partial(shift_body, src=raw_c, dst=row_c), 0)
        lax.fori_loop(0, B_PER_W // L,
                      functools.partial(shift_body, src=raw_x, dst=row_x), 0)
        lax.fori_loop(0, NEG_PER_W // L,
                      functools.partial(shift_body, src=raw_n, dst=row_n), 0)

        def colbase(raw_vec):
            return jnp.where(raw_vec >= SPLIT, EMBED, 0)

        def issue(step, rows_c, rows_x, rows_n, sem):
            cb = step * CHUNK
            nb = step * NEG_ROWS
            pltpu.async_copy(wc_h.at[row_c.at[pl.ds(cb, CHUNK)]],
                             rows_c, sem)
            pltpu.async_copy(wx_h.at[row_x.at[pl.ds(cb, CHUNK)]],
                             rows_x, sem)
            off = 0
            for seg in NSEG:
                pltpu.async_copy(wx_h.at[row_n.at[pl.ds(nb + off, seg)]],
                                 rows_n.at[pl.ds(off, seg)], sem)
                off += seg

        def drain(rows_c, rows_x, rows_n, sem):
            pltpu.make_async_copy(wc_h.at[pl.ds(0, CHUNK)], rows_c,
                                  sem).wait()
            pltpu.make_async_copy(wc_h.at[pl.ds(0, CHUNK)], rows_x,
                                  sem).wait()
            off = 0
            for seg in NSEG:
                pltpu.make_async_copy(wc_h.at[pl.ds(0, seg)],
                                      rows_n.at[pl.ds(off, seg)],
                                      sem).wait()
                off += seg

        def compute(step, rows_c, rows_x, rows_n):
            loc16 = _iota16()
            tb = loc16 + step * CHUNK
            tb20 = tb * NUM_NEG
            rowb = loc16 * NUM_NEG
            colc = colbase(plsc.load_gather(raw_c, [tb]))
            KH = NUM_NEG // 2

            colx = colbase(plsc.load_gather(raw_x, [tb]))
            coln = [colbase(plsc.load_gather(raw_n, [tb20 + k]))
                    for k in range(KH)]
            rowk = [rowb + k for k in range(KH)]

            def body_a(dd, accs):
                v = plsc.load_gather(rows_c, [loc16, colc + dd])
                up = plsc.load_gather(rows_x, [loc16, colx + dd])
                new = [accs[0] + v * up]
                for k in range(KH):
                    un = plsc.load_gather(rows_n, [rowk[k], coln[k] + dd])
                    new.append(accs[k + 1] + v * un)
                return tuple(new)

            accs = lax.fori_loop(
                0, EMBED, body_a,
                tuple(jnp.zeros((L,), jnp.float32) for _ in range(KH + 1)))
            plsc.store_scatter(pos_v, [tb], accs[0])
            for k in range(KH):
                plsc.store_scatter(neg_v, [tb20 + k], accs[k + 1])

            coln2 = [colbase(plsc.load_gather(raw_n, [tb20 + KH + k]))
                     for k in range(KH)]
            rowk2 = [rowb + KH + k for k in range(KH)]

            def body_b(dd, accs):
                v = plsc.load_gather(rows_c, [loc16, colc + dd])
                new = []
                for k in range(KH):
                    un = plsc.load_gather(rows_n,
                                          [rowk2[k], coln2[k] + dd])
                    new.append(accs[k] + v * un)
                return tuple(new)

            accs = lax.fori_loop(
                0, EMBED, body_b,
                tuple(jnp.zeros((L,), jnp.float32) for _ in range(KH)))
            for k in range(KH):
                plsc.store_scatter(neg_v, [tb20 + KH + k], accs[k])

        issue(0, rows_cA, rows_xA, rows_nA, semA)

        def pair_body(i, _):
            sa = 2 * i
            issue(sa + 1, rows_cB, rows_xB, rows_nB, semB)
            drain(rows_cA, rows_xA, rows_nA, semA)
            compute(sa, rows_cA, rows_xA, rows_nA)

            @pl.when(i < NSTEPS // 2 - 1)
            def _():
                issue(sa + 2, rows_cA, rows_xA, rows_nA, semA)

            drain(rows_cB, rows_xB, rows_nB, semB)
            compute(sa + 1, rows_cB, rows_xB, rows_nB)
            return 0

        lax.fori_loop(0, NSTEPS // 2, pair_body, 0)
        pltpu.sync_copy(pos_v, pos_h.at[pl.ds(base, B_PER_W)])
        pltpu.sync_copy(neg_v, neg_out_h.at[pl.ds(base * NUM_NEG, NEG_PER_W)])

    return scores_kernel(center, context, neg_flat, wc2, wx2)


def _iota16():
    return lax.iota(jnp.int32, L)


def _loss_kernel(pos_ref, neg_ref, out_ref):
    def log_sigmoid(x):
        return jnp.minimum(x, 0.0) - jnp.log1p(jnp.exp(-jnp.abs(x)))

    total = (jnp.sum(log_sigmoid(pos_ref[...]))
             + jnp.sum(log_sigmoid(-neg_ref[...])))
    out_ref[0, 0] = -total / BATCH


def kernel(center, context, negatives, W_center, W_context):
    center = center.astype(jnp.int32)
    context = context.astype(jnp.int32)
    neg_flat = negatives.astype(jnp.int32).reshape(BATCH * NUM_NEG)
    wc2, wx2 = _transpose_pack(W_center.T, W_context.T)
    pos, neg = _sc_scores(center, context, neg_flat, wc2, wx2)
    loss = pl.pallas_call(
        _loss_kernel,
        out_shape=jax.ShapeDtypeStruct((1, 1), jnp.float32),
        in_specs=[
            pl.BlockSpec(memory_space=pltpu.VMEM),
            pl.BlockSpec(memory_space=pltpu.VMEM),
        ],
        out_specs=pl.BlockSpec(memory_space=pltpu.SMEM),
    )(pos.reshape(BATCH // 128, 128), neg.reshape(BATCH * NUM_NEG // 128, 128))
    return loss[0, 0]


# speedup vs baseline: 2.5845x; 1.5545x over previous
import functools

import jax
import jax.numpy as jnp
from jax import lax
from jax.experimental import pallas as pl
from jax.experimental.pallas import tpu as pltpu
from jax.experimental.pallas import tpu_sc as plsc

VOCAB = 1000000
EMBED = 64
BATCH = 16384
NUM_NEG = 20

NC, NS, L = 2, 16, 16
NW = NC * NS
B_PER_W = BATCH // NW
PHYS = 2 * EMBED
CHUNK = 16
NSTEPS = B_PER_W // CHUNK
NEG_ROWS = CHUNK * NUM_NEG
NSEG = (128, 128, 64)
NEG_PER_W = B_PER_W * NUM_NEG

TBLK = 2048
NTBLK = 245
SPLIT = NTBLK * TBLK
NRBLK = (VOCAB - SPLIT + TBLK - 1) // TBLK + NTBLK


def _transpose_pack(wt_c, wt_x):
    def body(c1_ref, c2_ref, x1_ref, x2_ref, oc_ref, ox_ref):
        eye = (lax.broadcasted_iota(jnp.int32, (EMBED, EMBED), 0)
               == lax.broadcasted_iota(jnp.int32, (EMBED, EMBED), 1)
               ).astype(jnp.float32)

        def tr(ref):
            return lax.dot_general(ref[...], eye, (((0,), (0,)), ((), ())),
                                   preferred_element_type=jnp.float32)

        oc_ref[...] = jnp.concatenate([tr(c1_ref), tr(c2_ref)], axis=1)
        ox_ref[...] = jnp.concatenate([tr(x1_ref), tr(x2_ref)], axis=1)

    left = lambda b: (0, b)
    right = lambda b: (0, jnp.minimum(b + NTBLK, NRBLK - 1))
    in_spec = [pl.BlockSpec((EMBED, TBLK), m) for m in (left, right)] * 2
    out_spec = pl.BlockSpec((TBLK, PHYS), lambda b: (b, 0))
    return pl.pallas_call(
        body,
        grid=(NTBLK,),
        in_specs=in_spec,
        out_specs=[out_spec, out_spec],
        out_shape=[jax.ShapeDtypeStruct((SPLIT, PHYS), jnp.float32)] * 2,
    )(wt_c, wt_c, wt_x, wt_x)


def _sc_scores(center, context, neg_flat, wc2, wx2):
    mesh = plsc.VectorSubcoreMesh(core_axis_name="c", subcore_axis_name="s")

    @functools.partial(
        pl.kernel,
        out_type=(
            jax.ShapeDtypeStruct((BATCH,), jnp.float32),
            jax.ShapeDtypeStruct((BATCH * NUM_NEG,), jnp.float32),
        ),
        mesh=mesh,
        scratch_types=[
            pltpu.VMEM((B_PER_W,), jnp.int32),
            pltpu.VMEM((B_PER_W,), jnp.int32),
            pltpu.VMEM((NEG_PER_W,), jnp.int32),
            pltpu.VMEM((B_PER_W,), jnp.int32),
            pltpu.VMEM((B_PER_W,), jnp.int32),
            pltpu.VMEM((NEG_PER_W,), jnp.int32),
            pltpu.VMEM((CHUNK, PHYS), jnp.float32),
            pltpu.VMEM((CHUNK, PHYS), jnp.float32),
            pltpu.VMEM((NEG_ROWS, PHYS), jnp.float32),
            pltpu.VMEM((CHUNK, PHYS), jnp.float32),
            pltpu.VMEM((CHUNK, PHYS), jnp.float32),
            pltpu.VMEM((NEG_ROWS, PHYS), jnp.float32),
            pltpu.VMEM((B_PER_W,), jnp.float32),
            pltpu.VMEM((NEG_PER_W,), jnp.float32),
            pltpu.SemaphoreType.DMA,
            pltpu.SemaphoreType.DMA,
        ],
        compiler_params=pltpu.CompilerParams(
            needs_layout_passes=False, use_tc_tiling_on_sc=True),
    )
    def scores_kernel(center_h, context_h, neg_h, wc_h, wx_h,
                      pos_h, neg_out_h,
                      raw_c, raw_x, raw_n, row_c, row_x, row_n,
                      rows_cA, rows_xA, rows_nA, rows_cB, rows_xB, rows_nB,
                      pos_v, neg_v, semA, semB):
        wid = lax.axis_index("s") * NC + lax.axis_index("c")
        base = wid * B_PER_W

        pltpu.sync_copy(center_h.at[pl.ds(base, B_PER_W)], raw_c)
        pltpu.sync_copy(context_h.at[pl.ds(base, B_PER_W)], raw_x)
        pltpu.sync_copy(neg_h.at[pl.ds(base * NUM_NEG, NEG_PER_W)], raw_n)

        def shift_body(i, _, src, dst):
            v16 = i * L + _iota16()
            x = plsc.load_gather(src, [v16])
            row = jnp.where(x >= SPLIT, x - SPLIT, x)
            plsc.store_scatter(dst, [v16], row)
            return 0

        lax.fori_loop(0, B_PER_W // L,
                      functools.partial(shift_body, src=raw_c, dst=row_c), 0)
        lax.fori_loop(0, B_PER_W // L,
                      functools.partial(shift_body, src=raw_x, dst=row_x), 0)
        lax.fori_loop(0, NEG_PER_W // L,
                      functools.partial(shift_body, src=raw_n, dst=row_n), 0)

        def colbase(raw_vec):
            return jnp.where(raw_vec >= SPLIT, EMBED, 0)

        def issue(step, rows_c, rows_x, rows_n, sem):
            cb = step * CHUNK
            nb = step * NEG_ROWS
            pltpu.async_copy(wc_h.at[row_c.at[pl.ds(cb, CHUNK)]],
                             rows_c, sem)
            pltpu.async_copy(wx_h.at[row_x.at[pl.ds(cb, CHUNK)]],
                             rows_x, sem)
            off = 0
            for seg in NSEG:
                pltpu.async_copy(wx_h.at[row_n.at[pl.ds(nb + off, seg)]],
                                 rows_n.at[pl.ds(off, seg)], sem)
                off += seg

        def drain(rows_c, rows_x, rows_n, sem):
            pltpu.make_async_copy(wc_h.at[pl.ds(0, CHUNK)], rows_c,
                                  sem).wait()
            pltpu.make_async_copy(wc_h.at[pl.ds(0, CHUNK)], rows_x,
                                  sem).wait()
            off = 0
            for seg in NSEG:
                pltpu.make_async_copy(wc_h.at[pl.ds(0, seg)],
                                      rows_n.at[pl.ds(off, seg)],
                                      sem).wait()
                off += seg

        def compute(step, rows_c, rows_x, rows_n):
            loc16 = _iota16()
            tb = loc16 + step * CHUNK
            tb20 = tb * NUM_NEG
            rowb = loc16 * NUM_NEG
            colc = colbase(plsc.load_gather(raw_c, [tb]))
            KH = NUM_NEG // 2

            colx = colbase(plsc.load_gather(raw_x, [tb]))
            coln = [colbase(plsc.load_gather(raw_n, [tb20 + k]))
                    for k in range(KH)]
            rowk = [rowb + k for k in range(KH)]

            def body_a(dd, accs):
                wrap = (dd + loc16) & (EMBED - 1)
                v = plsc.load_gather(rows_c, [loc16, colc + wrap])
                up = plsc.load_gather(rows_x, [loc16, colx + wrap])
                new = [accs[0] + v * up]
                for k in range(KH):
                    un = plsc.load_gather(rows_n, [rowk[k], coln[k] + wrap])
                    new.append(accs[k + 1] + v * un)
                return tuple(new)

            accs = lax.fori_loop(
                0, EMBED, body_a,
                tuple(jnp.zeros((L,), jnp.float32) for _ in range(KH + 1)))
            plsc.store_scatter(pos_v, [tb], accs[0])
            for k in range(KH):
                plsc.store_scatter(neg_v, [tb20 + k], accs[k + 1])

            coln2 = [colbase(plsc.load_gather(raw_n, [tb20 + KH + k]))
                     for k in range(KH)]
            rowk2 = [rowb + KH + k for k in range(KH)]

            def body_b(dd, accs):
                wrap = (dd + loc16) & (EMBED - 1)
                v = plsc.load_gather(rows_c, [loc16, colc + wrap])
                new = []
                for k in range(KH):
                    un = plsc.load_gather(rows_n,
                                          [rowk2[k], coln2[k] + wrap])
                    new.append(accs[k] + v * un)
                return tuple(new)

            accs = lax.fori_loop(
                0, EMBED, body_b,
                tuple(jnp.zeros((L,), jnp.float32) for _ in range(KH)))
            for k in range(KH):
                plsc.store_scatter(neg_v, [tb20 + KH + k], accs[k])

        issue(0, rows_cA, rows_xA, rows_nA, semA)

        def pair_body(i, _):
            sa = 2 * i
            issue(sa + 1, rows_cB, rows_xB, rows_nB, semB)
            drain(rows_cA, rows_xA, rows_nA, semA)
            compute(sa, rows_cA, rows_xA, rows_nA)

            @pl.when(i < NSTEPS // 2 - 1)
            def _():
                issue(sa + 2, rows_cA, rows_xA, rows_nA, semA)

            drain(rows_cB, rows_xB, rows_nB, semB)
            compute(sa + 1, rows_cB, rows_xB, rows_nB)
            return 0

        lax.fori_loop(0, NSTEPS // 2, pair_body, 0)
        pltpu.sync_copy(pos_v, pos_h.at[pl.ds(base, B_PER_W)])
        pltpu.sync_copy(neg_v, neg_out_h.at[pl.ds(base * NUM_NEG, NEG_PER_W)])

    return scores_kernel(center, context, neg_flat, wc2, wx2)


def _iota16():
    return lax.iota(jnp.int32, L)


def _loss_kernel(pos_ref, neg_ref, out_ref):
    def log_sigmoid(x):
        return jnp.minimum(x, 0.0) - jnp.log1p(jnp.exp(-jnp.abs(x)))

    total = (jnp.sum(log_sigmoid(pos_ref[...]))
             + jnp.sum(log_sigmoid(-neg_ref[...])))
    out_ref[0, 0] = -total / BATCH


def kernel(center, context, negatives, W_center, W_context):
    center = center.astype(jnp.int32)
    context = context.astype(jnp.int32)
    neg_flat = negatives.astype(jnp.int32).reshape(BATCH * NUM_NEG)
    wc2, wx2 = _transpose_pack(W_center.T, W_context.T)
    pos, neg = _sc_scores(center, context, neg_flat, wc2, wx2)
    loss = pl.pallas_call(
        _loss_kernel,
        out_shape=jax.ShapeDtypeStruct((1, 1), jnp.float32),
        in_specs=[
            pl.BlockSpec(memory_space=pltpu.VMEM),
            pl.BlockSpec(memory_space=pltpu.VMEM),
        ],
        out_specs=pl.BlockSpec(memory_space=pltpu.SMEM),
    )(pos.reshape(BATCH // 128, 128), neg.reshape(BATCH * NUM_NEG // 128, 128))
    return loss[0, 0]


# speedup vs baseline: 2.9175x; 1.1288x over previous
import functools

import jax
import jax.numpy as jnp
from jax import lax
from jax.experimental import pallas as pl
from jax.experimental.pallas import tpu as pltpu
from jax.experimental.pallas import tpu_sc as plsc

VOCAB = 1000000
EMBED = 64
BATCH = 16384
NUM_NEG = 20

NC, NS, L = 2, 16, 16
NW = NC * NS
B_PER_W = BATCH // NW
PHYS = 2 * EMBED
CHUNK = 16
NSTEPS = B_PER_W // CHUNK
NEG_ROWS = CHUNK * NUM_NEG
NSEG = (128, 128, 64)
NEG_PER_W = B_PER_W * NUM_NEG

TBLK = 1024
NTBLK = 245
QUART = NTBLK * TBLK
NCOLB = (VOCAB + TBLK - 1) // TBLK - 1


def _transpose_pack(wt_c, wt_x):
    def body(c0, c1, c2, c3, x0, x1, x2, x3, oc_ref, ox_ref):
        eye = (lax.broadcasted_iota(jnp.int32, (EMBED, EMBED), 0)
               == lax.broadcasted_iota(jnp.int32, (EMBED, EMBED), 1)
               ).astype(jnp.float32)

        def tr(ref):
            return lax.dot_general(ref[...], eye, (((0,), (0,)), ((), ())),
                                   preferred_element_type=jnp.float32)

        def bf16_bits(x):
            rounded = x.astype(jnp.bfloat16).astype(jnp.float32)
            return lax.bitcast_convert_type(rounded, jnp.uint32)

        def pack(lo, hi):
            word = ((bf16_bits(lo) >> 16)
                    | (bf16_bits(hi) & jnp.uint32(0xFFFF0000)))
            return lax.bitcast_convert_type(word, jnp.int32)

        def packed(r0, r1, r2, r3):
            return jnp.concatenate(
                [pack(tr(r0), tr(r2)), pack(tr(r1), tr(r3))], axis=1)

        oc_ref[...] = packed(c0, c1, c2, c3)
        ox_ref[...] = packed(x0, x1, x2, x3)

    def qmap(qs):
        if qs == 3 * NTBLK:
            return lambda b: (0, jnp.minimum(qs + b, NCOLB))
        return lambda b: (0, qs + b)

    in_spec = [pl.BlockSpec((EMBED, TBLK), qmap(q * NTBLK))
               for q in range(4)] * 2
    out_spec = pl.BlockSpec((TBLK, PHYS), lambda b: (b, 0))
    return pl.pallas_call(
        body,
        grid=(NTBLK,),
        in_specs=in_spec,
        out_specs=[out_spec, out_spec],
        out_shape=[jax.ShapeDtypeStruct((QUART, PHYS), jnp.int32)] * 2,
    )(wt_c, wt_c, wt_c, wt_c, wt_x, wt_x, wt_x, wt_x)


def _sc_scores(center, context, neg_flat, wc2, wx2):
    mesh = plsc.VectorSubcoreMesh(core_axis_name="c", subcore_axis_name="s")

    @functools.partial(
        pl.kernel,
        out_type=(
            jax.ShapeDtypeStruct((BATCH,), jnp.float32),
            jax.ShapeDtypeStruct((BATCH * NUM_NEG,), jnp.float32),
        ),
        mesh=mesh,
        scratch_types=[
            pltpu.VMEM((B_PER_W,), jnp.int32),
            pltpu.VMEM((B_PER_W,), jnp.int32),
            pltpu.VMEM((NEG_PER_W,), jnp.int32),
            pltpu.VMEM((B_PER_W,), jnp.int32),
            pltpu.VMEM((B_PER_W,), jnp.int32),
            pltpu.VMEM((NEG_PER_W,), jnp.int32),
            pltpu.VMEM((CHUNK, PHYS), jnp.int32),
            pltpu.VMEM((CHUNK, PHYS), jnp.int32),
            pltpu.VMEM((NEG_ROWS, PHYS), jnp.int32),
            pltpu.VMEM((CHUNK, PHYS), jnp.int32),
            pltpu.VMEM((CHUNK, PHYS), jnp.int32),
            pltpu.VMEM((NEG_ROWS, PHYS), jnp.int32),
            pltpu.VMEM((B_PER_W,), jnp.float32),
            pltpu.VMEM((NEG_PER_W,), jnp.float32),
            pltpu.SemaphoreType.DMA,
            pltpu.SemaphoreType.DMA,
        ],
        compiler_params=pltpu.CompilerParams(
            needs_layout_passes=False, use_tc_tiling_on_sc=True),
    )
    def scores_kernel(center_h, context_h, neg_h, wc_h, wx_h,
                      pos_h, neg_out_h,
                      raw_c, raw_x, raw_n, row_c, row_x, row_n,
                      rows_cA, rows_xA, rows_nA, rows_cB, rows_xB, rows_nB,
                      pos_v, neg_v, semA, semB):
        wid = lax.axis_index("s") * NC + lax.axis_index("c")
        base = wid * B_PER_W

        pltpu.sync_copy(center_h.at[pl.ds(base, B_PER_W)], raw_c)
        pltpu.sync_copy(context_h.at[pl.ds(base, B_PER_W)], raw_x)
        pltpu.sync_copy(neg_h.at[pl.ds(base * NUM_NEG, NEG_PER_W)], raw_n)

        def quad_of(x):
            return ((x >= QUART).astype(jnp.int32)
                    + (x >= 2 * QUART).astype(jnp.int32)
                    + (x >= 3 * QUART).astype(jnp.int32))

        def shift_body(i, _, src, dst):
            v16 = i * L + _iota16()
            x = plsc.load_gather(src, [v16])
            plsc.store_scatter(dst, [v16], x - quad_of(x) * QUART)
            return 0

        lax.fori_loop(0, B_PER_W // L,
                      functools.partial(shift_body, src=raw_c, dst=row_c), 0)
        lax.fori_loop(0, B_PER_W // L,
                      functools.partial(shift_body, src=raw_x, dst=row_x), 0)
        lax.fori_loop(0, NEG_PER_W // L,
                      functools.partial(shift_body, src=raw_n, dst=row_n), 0)

        def wordsel(raw_vec):
            quad = quad_of(raw_vec)
            wb = (quad & 1) << 6
            lsh = jnp.where(quad >= 2, 0, 16).astype(jnp.uint32)
            return wb, lsh

        def unpack(word, lsh):
            u = plsc.bitcast(word, jnp.uint32)
            bits = (u << lsh) & jnp.uint32(0xFFFF0000)
            return plsc.bitcast(bits, jnp.float32)

        def issue(step, rows_c, rows_x, rows_n, sem):
            cb = step * CHUNK
            nb = step * NEG_ROWS
            pltpu.async_copy(wc_h.at[row_c.at[pl.ds(cb, CHUNK)]],
                             rows_c, sem)
            pltpu.async_copy(wx_h.at[row_x.at[pl.ds(cb, CHUNK)]],
                             rows_x, sem)
            off = 0
            for seg in NSEG:
                pltpu.async_copy(wx_h.at[row_n.at[pl.ds(nb + off, seg)]],
                                 rows_n.at[pl.ds(off, seg)], sem)
                off += seg

        def drain(rows_c, rows_x, rows_n, sem):
            pltpu.make_async_copy(wc_h.at[pl.ds(0, CHUNK)], rows_c,
                                  sem).wait()
            pltpu.make_async_copy(wc_h.at[pl.ds(0, CHUNK)], rows_x,
                                  sem).wait()
            off = 0
            for seg in NSEG:
                pltpu.make_async_copy(wc_h.at[pl.ds(0, seg)],
                                      rows_n.at[pl.ds(off, seg)],
                                      sem).wait()
                off += seg

        def compute(step, rows_c, rows_x, rows_n):
          for g in range(CHUNK // L):
            loc16 = _iota16() + g * L
            tb = loc16 + step * CHUNK
            tb20 = tb * NUM_NEG
            rowb = loc16 * NUM_NEG
            wbc, lshc = wordsel(plsc.load_gather(raw_c, [tb]))
            KH = NUM_NEG // 2

            wbx, lshx = wordsel(plsc.load_gather(raw_x, [tb]))
            seln = [wordsel(plsc.load_gather(raw_n, [tb20 + k]))
                    for k in range(KH)]
            rowk = [rowb + k for k in range(KH)]

            def body_a(dd, accs, loc16=loc16, wbc=wbc, lshc=lshc,
                       wbx=wbx, lshx=lshx, seln=seln, rowk=rowk):
                wrap = (dd + loc16) & (EMBED - 1)
                v = unpack(plsc.load_gather(rows_c, [loc16, wbc + wrap]),
                           lshc)
                up = unpack(plsc.load_gather(rows_x, [loc16, wbx + wrap]),
                            lshx)
                new = [accs[0] + v * up]
                for k in range(KH):
                    un = unpack(plsc.load_gather(
                        rows_n, [rowk[k], seln[k][0] + wrap]), seln[k][1])
                    new.append(accs[k + 1] + v * un)
                return tuple(new)

            accs = lax.fori_loop(
                0, EMBED, body_a,
                tuple(jnp.zeros((L,), jnp.float32) for _ in range(KH + 1)))
            plsc.store_scatter(pos_v, [tb], accs[0])
            for k in range(KH):
                plsc.store_scatter(neg_v, [tb20 + k], accs[k + 1])

            seln2 = [wordsel(plsc.load_gather(raw_n, [tb20 + KH + k]))
                     for k in range(KH)]
            rowk2 = [rowb + KH + k for k in range(KH)]

            def body_b(dd, accs, loc16=loc16, wbc=wbc, lshc=lshc,
                       seln2=seln2, rowk2=rowk2):
                wrap = (dd + loc16) & (EMBED - 1)
                v = unpack(plsc.load_gather(rows_c, [loc16, wbc + wrap]),
                           lshc)
                new = []
                for k in range(KH):
                    un = unpack(plsc.load_gather(
                        rows_n, [rowk2[k], seln2[k][0] + wrap]),
                        seln2[k][1])
                    new.append(accs[k] + v * un)
                return tuple(new)

            accs = lax.fori_loop(
                0, EMBED, body_b,
                tuple(jnp.zeros((L,), jnp.float32) for _ in range(KH)))
            for k in range(KH):
                plsc.store_scatter(neg_v, [tb20 + KH + k], accs[k])

        issue(0, rows_cA, rows_xA, rows_nA, semA)

        def pair_body(i, _):
            sa = 2 * i
            issue(sa + 1, rows_cB, rows_xB, rows_nB, semB)
            drain(rows_cA, rows_xA, rows_nA, semA)
            compute(sa, rows_cA, rows_xA, rows_nA)

            @pl.when(i < NSTEPS // 2 - 1)
            def _():
                issue(sa + 2, rows_cA, rows_xA, rows_nA, semA)

            drain(rows_cB, rows_xB, rows_nB, semB)
            compute(sa + 1, rows_cB, rows_xB, rows_nB)
            return 0

        lax.fori_loop(0, NSTEPS // 2, pair_body, 0)
        pltpu.sync_copy(pos_v, pos_h.at[pl.ds(base, B_PER_W)])
        pltpu.sync_copy(neg_v, neg_out_h.at[pl.ds(base * NUM_NEG, NEG_PER_W)])

    return scores_kernel(center, context, neg_flat, wc2, wx2)


def _iota16():
    return lax.iota(jnp.int32, L)


def _loss_kernel(pos_ref, neg_ref, out_ref):
    def log_sigmoid(x):
        return jnp.minimum(x, 0.0) - jnp.log1p(jnp.exp(-jnp.abs(x)))

    total = (jnp.sum(log_sigmoid(pos_ref[...]))
             + jnp.sum(log_sigmoid(-neg_ref[...])))
    out_ref[0, 0] = -total / BATCH


def kernel(center, context, negatives, W_center, W_context):
    center = center.astype(jnp.int32)
    context = context.astype(jnp.int32)
    neg_flat = negatives.astype(jnp.int32).reshape(BATCH * NUM_NEG)
    wc2, wx2 = _transpose_pack(W_center.T, W_context.T)
    pos, neg = _sc_scores(center, context, neg_flat, wc2, wx2)
    loss = pl.pallas_call(
        _loss_kernel,
        out_shape=jax.ShapeDtypeStruct((1, 1), jnp.float32),
        in_specs=[
            pl.BlockSpec(memory_space=pltpu.VMEM),
            pl.BlockSpec(memory_space=pltpu.VMEM),
        ],
        out_specs=pl.BlockSpec(memory_space=pltpu.SMEM),
    )(pos.reshape(BATCH // 128, 128), neg.reshape(BATCH * NUM_NEG // 128, 128))
    return loss[0, 0]


# speedup vs baseline: 3.4418x; 1.1797x over previous
import functools

import jax
import jax.numpy as jnp
from jax import lax
from jax.experimental import pallas as pl
from jax.experimental.pallas import tpu as pltpu
from jax.experimental.pallas import tpu_sc as plsc

VOCAB = 1000000
EMBED = 64
BATCH = 16384
NUM_NEG = 20

NC, NS, L = 2, 16, 16
NW = NC * NS
B_PER_W = BATCH // NW
PHYS = 2 * EMBED
CHUNK = 16
NSTEPS = B_PER_W // CHUNK
NEG_ROWS = CHUNK * NUM_NEG
NSEG = (128, 128, 64)
NEG_PER_W = B_PER_W * NUM_NEG

TBLK = 1280
NTBLK = 196
QUART = NTBLK * TBLK
NCOLB = (VOCAB + TBLK - 1) // TBLK - 1


def _transpose_pack(wt_c, wt_x):
    def body(c0, c1, c2, c3, x0, x1, x2, x3, oc_ref, ox_ref):
        eye = (lax.broadcasted_iota(jnp.int32, (EMBED, EMBED), 0)
               == lax.broadcasted_iota(jnp.int32, (EMBED, EMBED), 1)
               ).astype(jnp.bfloat16)

        def tr(ref):
            return lax.dot_general(ref[...].astype(jnp.bfloat16), eye,
                                   (((0,), (0,)), ((), ())),
                                   preferred_element_type=jnp.float32)

        def pack(lo, hi):
            lo_bits = lax.bitcast_convert_type(lo, jnp.uint32)
            hi_bits = lax.bitcast_convert_type(hi, jnp.uint32)
            word = (lo_bits >> 16) | (hi_bits & jnp.uint32(0xFFFF0000))
            return lax.bitcast_convert_type(word, jnp.int32)

        def packed(r0, r1, r2, r3):
            return jnp.concatenate(
                [pack(tr(r0), tr(r2)), pack(tr(r1), tr(r3))], axis=1)

        oc_ref[...] = packed(c0, c1, c2, c3)
        ox_ref[...] = packed(x0, x1, x2, x3)

    def qmap(qs):
        if qs == 3 * NTBLK:
            return lambda b: (0, jnp.minimum(qs + b, NCOLB))
        return lambda b: (0, qs + b)

    in_spec = [pl.BlockSpec((EMBED, TBLK), qmap(q * NTBLK))
               for q in range(4)] * 2
    out_spec = pl.BlockSpec((TBLK, PHYS), lambda b: (b, 0))
    return pl.pallas_call(
        body,
        grid=(NTBLK,),
        in_specs=in_spec,
        out_specs=[out_spec, out_spec],
        out_shape=[jax.ShapeDtypeStruct((QUART, PHYS), jnp.int32)] * 2,
    )(wt_c, wt_c, wt_c, wt_c, wt_x, wt_x, wt_x, wt_x)


def _sc_scores(center, context, neg_flat, wc2, wx2):
    mesh = plsc.VectorSubcoreMesh(core_axis_name="c", subcore_axis_name="s")

    @functools.partial(
        pl.kernel,
        out_type=(
            jax.ShapeDtypeStruct((BATCH,), jnp.float32),
            jax.ShapeDtypeStruct((BATCH * NUM_NEG,), jnp.float32),
        ),
        mesh=mesh,
        scratch_types=[
            pltpu.VMEM((B_PER_W,), jnp.int32),
            pltpu.VMEM((B_PER_W,), jnp.int32),
            pltpu.VMEM((NEG_PER_W,), jnp.int32),
            pltpu.VMEM((B_PER_W,), jnp.int32),
            pltpu.VMEM((B_PER_W,), jnp.int32),
            pltpu.VMEM((NEG_PER_W,), jnp.int32),
            pltpu.VMEM((CHUNK, PHYS), jnp.int32),
            pltpu.VMEM((CHUNK, PHYS), jnp.int32),
            pltpu.VMEM((NEG_ROWS, PHYS), jnp.int32),
            pltpu.VMEM((CHUNK, PHYS), jnp.int32),
            pltpu.VMEM((CHUNK, PHYS), jnp.int32),
            pltpu.VMEM((NEG_ROWS, PHYS), jnp.int32),
            pltpu.VMEM((B_PER_W,), jnp.float32),
            pltpu.VMEM((NEG_PER_W,), jnp.float32),
            pltpu.SemaphoreType.DMA,
            pltpu.SemaphoreType.DMA,
        ],
        compiler_params=pltpu.CompilerParams(
            needs_layout_passes=False, use_tc_tiling_on_sc=True),
    )
    def scores_kernel(center_h, context_h, neg_h, wc_h, wx_h,
                      pos_h, neg_out_h,
                      raw_c, raw_x, raw_n, row_c, row_x, row_n,
                      rows_cA, rows_xA, rows_nA, rows_cB, rows_xB, rows_nB,
                      pos_v, neg_v, semA, semB):
        wid = lax.axis_index("s") * NC + lax.axis_index("c")
        base = wid * B_PER_W

        pltpu.sync_copy(center_h.at[pl.ds(base, B_PER_W)], raw_c)
        pltpu.sync_copy(context_h.at[pl.ds(base, B_PER_W)], raw_x)
        pltpu.sync_copy(neg_h.at[pl.ds(base * NUM_NEG, NEG_PER_W)], raw_n)

        def quad_of(x):
            return ((x >= QUART).astype(jnp.int32)
                    + (x >= 2 * QUART).astype(jnp.int32)
                    + (x >= 3 * QUART).astype(jnp.int32))

        def shift_body(i, _, src, dst):
            v16 = i * L + _iota16()
            x = plsc.load_gather(src, [v16])
            plsc.store_scatter(dst, [v16], x - quad_of(x) * QUART)
            return 0

        lax.fori_loop(0, B_PER_W // L,
                      functools.partial(shift_body, src=raw_c, dst=row_c), 0)
        lax.fori_loop(0, B_PER_W // L,
                      functools.partial(shift_body, src=raw_x, dst=row_x), 0)
        lax.fori_loop(0, NEG_PER_W // L,
                      functools.partial(shift_body, src=raw_n, dst=row_n), 0)

        def wordsel(raw_vec):
            quad = quad_of(raw_vec)
            wb = (quad & 1) << 6
            lsh = jnp.where(quad >= 2, 0, 16).astype(jnp.uint32)
            return wb, lsh

        def unpack(word, lsh):
            u = plsc.bitcast(word, jnp.uint32)
            bits = (u << lsh) & jnp.uint32(0xFFFF0000)
            return plsc.bitcast(bits, jnp.float32)

        def issue(step, rows_c, rows_x, rows_n, sem):
            cb = step * CHUNK
            nb = step * NEG_ROWS
            pltpu.async_copy(wc_h.at[row_c.at[pl.ds(cb, CHUNK)]],
                             rows_c, sem)
            pltpu.async_copy(wx_h.at[row_x.at[pl.ds(cb, CHUNK)]],
                             rows_x, sem)
            off = 0
            for seg in NSEG:
                pltpu.async_copy(wx_h.at[row_n.at[pl.ds(nb + off, seg)]],
                                 rows_n.at[pl.ds(off, seg)], sem)
                off += seg

        def drain(rows_c, rows_x, rows_n, sem):
            pltpu.make_async_copy(wc_h.at[pl.ds(0, CHUNK)], rows_c,
                                  sem).wait()
            pltpu.make_async_copy(wc_h.at[pl.ds(0, CHUNK)], rows_x,
                                  sem).wait()
            off = 0
            for seg in NSEG:
                pltpu.make_async_copy(wc_h.at[pl.ds(0, seg)],
                                      rows_n.at[pl.ds(off, seg)],
                                      sem).wait()
                off += seg

        def compute(step, rows_c, rows_x, rows_n):
          for g in range(CHUNK // L):
            loc16 = _iota16() + g * L
            tb = loc16 + step * CHUNK
            tb20 = tb * NUM_NEG
            rowb = loc16 * NUM_NEG
            wbc, lshc = wordsel(plsc.load_gather(raw_c, [tb]))
            KH = NUM_NEG // 2

            wbx, lshx = wordsel(plsc.load_gather(raw_x, [tb]))
            seln = [wordsel(plsc.load_gather(raw_n, [tb20 + k]))
                    for k in range(KH)]
            rowk = [rowb + k for k in range(KH)]

            def body_a(dd, accs, loc16=loc16, wbc=wbc, lshc=lshc,
                       wbx=wbx, lshx=lshx, seln=seln, rowk=rowk):
                wrap = (dd + loc16) & (EMBED - 1)
                v = unpack(plsc.load_gather(rows_c, [loc16, wbc + wrap]),
                           lshc)
                up = unpack(plsc.load_gather(rows_x, [loc16, wbx + wrap]),
                            lshx)
                new = [accs[0] + v * up]
                for k in range(KH):
                    un = unpack(plsc.load_gather(
                        rows_n, [rowk[k], seln[k][0] + wrap]), seln[k][1])
                    new.append(accs[k + 1] + v * un)
                return tuple(new)

            accs = lax.fori_loop(
                0, EMBED, body_a,
                tuple(jnp.zeros((L,), jnp.float32) for _ in range(KH + 1)))
            plsc.store_scatter(pos_v, [tb], accs[0])
            for k in range(KH):
                plsc.store_scatter(neg_v, [tb20 + k], accs[k + 1])

            seln2 = [wordsel(plsc.load_gather(raw_n, [tb20 + KH + k]))
                     for k in range(KH)]
            rowk2 = [rowb + KH + k for k in range(KH)]

            def body_b(dd, accs, loc16=loc16, wbc=wbc, lshc=lshc,
                       seln2=seln2, rowk2=rowk2):
                wrap = (dd + loc16) & (EMBED - 1)
                v = unpack(plsc.load_gather(rows_c, [loc16, wbc + wrap]),
                           lshc)
                new = []
                for k in range(KH):
                    un = unpack(plsc.load_gather(
                        rows_n, [rowk2[k], seln2[k][0] + wrap]),
                        seln2[k][1])
                    new.append(accs[k] + v * un)
                return tuple(new)

            accs = lax.fori_loop(
                0, EMBED, body_b,
                tuple(jnp.zeros((L,), jnp.float32) for _ in range(KH)))
            for k in range(KH):
                plsc.store_scatter(neg_v, [tb20 + KH + k], accs[k])

        issue(0, rows_cA, rows_xA, rows_nA, semA)

        def pair_body(i, _):
            sa = 2 * i
            issue(sa + 1, rows_cB, rows_xB, rows_nB, semB)
            drain(rows_cA, rows_xA, rows_nA, semA)
            compute(sa, rows_cA, rows_xA, rows_nA)

            @pl.when(i < NSTEPS // 2 - 1)
            def _():
                issue(sa + 2, rows_cA, rows_xA, rows_nA, semA)

            drain(rows_cB, rows_xB, rows_nB, semB)
            compute(sa + 1, rows_cB, rows_xB, rows_nB)
            return 0

        lax.fori_loop(0, NSTEPS // 2, pair_body, 0)
        pltpu.sync_copy(pos_v, pos_h.at[pl.ds(base, B_PER_W)])
        pltpu.sync_copy(neg_v, neg_out_h.at[pl.ds(base * NUM_NEG, NEG_PER_W)])

    return scores_kernel(center, context, neg_flat, wc2, wx2)


def _iota16():
    return lax.iota(jnp.int32, L)


def _loss_kernel(pos_ref, neg_ref, out_ref):
    def log_sigmoid(x):
        return jnp.minimum(x, 0.0) - jnp.log1p(jnp.exp(-jnp.abs(x)))

    total = (jnp.sum(log_sigmoid(pos_ref[...]))
             + jnp.sum(log_sigmoid(-neg_ref[...])))
    out_ref[0, 0] = -total / BATCH


def kernel(center, context, negatives, W_center, W_context):
    center = center.astype(jnp.int32)
    context = context.astype(jnp.int32)
    neg_flat = negatives.astype(jnp.int32).reshape(BATCH * NUM_NEG)
    wc2, wx2 = _transpose_pack(W_center.T, W_context.T)
    pos, neg = _sc_scores(center, context, neg_flat, wc2, wx2)
    loss = pl.pallas_call(
        _loss_kernel,
        out_shape=jax.ShapeDtypeStruct((1, 1), jnp.float32),
        in_specs=[
            pl.BlockSpec(memory_space=pltpu.VMEM),
            pl.BlockSpec(memory_space=pltpu.VMEM),
        ],
        out_specs=pl.BlockSpec(memory_space=pltpu.SMEM),
    )(pos.reshape(BATCH // 128, 128), neg.reshape(BATCH * NUM_NEG // 128, 128))
    return loss[0, 0]


# speedup vs baseline: 3.5697x; 1.0372x over previous
import functools

import jax
import jax.numpy as jnp
from jax import lax
from jax.experimental import pallas as pl
from jax.experimental.pallas import tpu as pltpu
from jax.experimental.pallas import tpu_sc as plsc

VOCAB = 1000000
EMBED = 64
BATCH = 16384
NUM_NEG = 20

NC, NS, L = 2, 16, 16
NW = NC * NS
B_PER_W = BATCH // NW
PHYS = 2 * EMBED
CHUNK = 16
NSTEPS = B_PER_W // CHUNK
NEG_ROWS = CHUNK * NUM_NEG
NSEG = (128, 128, 64)
NEG_PER_W = B_PER_W * NUM_NEG

TBLK = 1280
NTBLK = 196
QUART = NTBLK * TBLK
NCOLB = (VOCAB + TBLK - 1) // TBLK - 1


def _transpose_pack(wt_c, wt_x):
    def body(c0, c1, c2, c3, x0, x1, x2, x3, oc_ref, ox_ref):
        eye = (lax.broadcasted_iota(jnp.int32, (EMBED, EMBED), 0)
               == lax.broadcasted_iota(jnp.int32, (EMBED, EMBED), 1)
               ).astype(jnp.bfloat16)

        def tr(ref):
            return lax.dot_general(ref[...].astype(jnp.bfloat16), eye,
                                   (((0,), (0,)), ((), ())),
                                   preferred_element_type=jnp.float32)

        def pack(lo, hi):
            lo_bits = lax.bitcast_convert_type(lo, jnp.uint32)
            hi_bits = lax.bitcast_convert_type(hi, jnp.uint32)
            word = (lo_bits >> 16) | (hi_bits & jnp.uint32(0xFFFF0000))
            return lax.bitcast_convert_type(word, jnp.int32)

        def packed(r0, r1, r2, r3):
            return jnp.concatenate(
                [pack(tr(r0), tr(r2)), pack(tr(r1), tr(r3))], axis=1)

        oc_ref[...] = packed(c0, c1, c2, c3)
        ox_ref[...] = packed(x0, x1, x2, x3)

    def qmap(qs):
        if qs == 3 * NTBLK:
            return lambda b: (0, jnp.minimum(qs + b, NCOLB))
        return lambda b: (0, qs + b)

    in_spec = [pl.BlockSpec((EMBED, TBLK), qmap(q * NTBLK))
               for q in range(4)] * 2
    out_spec = pl.BlockSpec((TBLK, PHYS), lambda b: (b, 0))
    return pl.pallas_call(
        body,
        grid=(NTBLK,),
        in_specs=in_spec,
        out_specs=[out_spec, out_spec],
        out_shape=[jax.ShapeDtypeStruct((QUART, PHYS), jnp.int32)] * 2,
    )(wt_c, wt_c, wt_c, wt_c, wt_x, wt_x, wt_x, wt_x)


def _sc_scores(center, context, neg_flat, wc2, wx2):
    mesh = plsc.VectorSubcoreMesh(core_axis_name="c", subcore_axis_name="s")

    @functools.partial(
        pl.kernel,
        out_type=(
            jax.ShapeDtypeStruct((BATCH,), jnp.float32),
            jax.ShapeDtypeStruct((BATCH * NUM_NEG,), jnp.float32),
        ),
        mesh=mesh,
        scratch_types=[
            pltpu.VMEM((B_PER_W,), jnp.int32),
            pltpu.VMEM((B_PER_W,), jnp.int32),
            pltpu.VMEM((NUM_NEG, B_PER_W), jnp.int32),
            pltpu.VMEM((B_PER_W,), jnp.int32),
            pltpu.VMEM((B_PER_W,), jnp.int32),
            pltpu.VMEM((NEG_PER_W,), jnp.int32),
            pltpu.VMEM((CHUNK, PHYS), jnp.int32),
            pltpu.VMEM((CHUNK, PHYS), jnp.int32),
            pltpu.VMEM((NEG_ROWS, PHYS), jnp.int32),
            pltpu.VMEM((CHUNK, PHYS), jnp.int32),
            pltpu.VMEM((CHUNK, PHYS), jnp.int32),
            pltpu.VMEM((NEG_ROWS, PHYS), jnp.int32),
            pltpu.VMEM((B_PER_W,), jnp.float32),
            pltpu.VMEM((NEG_PER_W,), jnp.float32),
            pltpu.SemaphoreType.DMA,
            pltpu.SemaphoreType.DMA,
        ],
        compiler_params=pltpu.CompilerParams(
            needs_layout_passes=False, use_tc_tiling_on_sc=True),
    )
    def scores_kernel(center_h, context_h, neg_h, wc_h, wx_h,
                      pos_h, neg_out_h,
                      raw_c, raw_x, raw_n, row_c, row_x, row_n,
                      rows_cA, rows_xA, rows_nA, rows_cB, rows_xB, rows_nB,
                      pos_v, neg_v, semA, semB):
        wid = lax.axis_index("s") * NC + lax.axis_index("c")
        base = wid * B_PER_W

        pltpu.sync_copy(center_h.at[pl.ds(base, B_PER_W)], raw_c)
        pltpu.sync_copy(context_h.at[pl.ds(base, B_PER_W)], raw_x)
        pltpu.sync_copy(neg_h.at[:, pl.ds(base, B_PER_W)], raw_n)

        def quad_of(x):
            return ((x >= QUART).astype(jnp.int32)
                    + (x >= 2 * QUART).astype(jnp.int32)
                    + (x >= 3 * QUART).astype(jnp.int32))

        def shift_body(i, _, src, dst):
            v16 = i * L + _iota16()
            x = plsc.load_gather(src, [v16])
            plsc.store_scatter(dst, [v16], x - quad_of(x) * QUART)
            return 0

        lax.fori_loop(0, B_PER_W // L,
                      functools.partial(shift_body, src=raw_c, dst=row_c), 0)
        lax.fori_loop(0, B_PER_W // L,
                      functools.partial(shift_body, src=raw_x, dst=row_x), 0)

        def neg_shift_body(j, _, k):
            v16 = j * L + _iota16()
            x = plsc.load_gather(raw_n, [jnp.full((L,), k, jnp.int32), v16])
            plsc.store_scatter(row_n, [v16 * NUM_NEG + k],
                               x - quad_of(x) * QUART)
            return 0

        for k in range(NUM_NEG):
            lax.fori_loop(0, B_PER_W // L,
                          functools.partial(neg_shift_body, k=k), 0)

        def wordsel(raw_vec):
            quad = quad_of(raw_vec)
            wb = (quad & 1) << 6
            lsh = jnp.where(quad >= 2, 0, 16).astype(jnp.uint32)
            return wb, lsh

        def unpack(word, lsh):
            u = plsc.bitcast(word, jnp.uint32)
            bits = (u << lsh) & jnp.uint32(0xFFFF0000)
            return plsc.bitcast(bits, jnp.float32)

        def issue(step, rows_c, rows_x, rows_n, sem):
            cb = step * CHUNK
            nb = step * NEG_ROWS
            pltpu.async_copy(wc_h.at[row_c.at[pl.ds(cb, CHUNK)]],
                             rows_c, sem)
            pltpu.async_copy(wx_h.at[row_x.at[pl.ds(cb, CHUNK)]],
                             rows_x, sem)
            off = 0
            for seg in NSEG:
                pltpu.async_copy(wx_h.at[row_n.at[pl.ds(nb + off, seg)]],
                                 rows_n.at[pl.ds(off, seg)], sem)
                off += seg

        def drain(rows_c, rows_x, rows_n, sem):
            pltpu.make_async_copy(wc_h.at[pl.ds(0, CHUNK)], rows_c,
                                  sem).wait()
            pltpu.make_async_copy(wc_h.at[pl.ds(0, CHUNK)], rows_x,
                                  sem).wait()
            off = 0
            for seg in NSEG:
                pltpu.make_async_copy(wc_h.at[pl.ds(0, seg)],
                                      rows_n.at[pl.ds(off, seg)],
                                      sem).wait()
                off += seg

        def compute(step, rows_c, rows_x, rows_n):
          for g in range(CHUNK // L):
            loc16 = _iota16() + g * L
            tb = loc16 + step * CHUNK
            tb20 = tb * NUM_NEG
            rowb = loc16 * NUM_NEG
            wbc, lshc = wordsel(plsc.load_gather(raw_c, [tb]))
            KH = NUM_NEG // 2

            wbx, lshx = wordsel(plsc.load_gather(raw_x, [tb]))
            seln = [wordsel(plsc.load_gather(
                        raw_n, [jnp.full((L,), k, jnp.int32), tb]))
                    for k in range(KH)]
            rowk = [rowb + k for k in range(KH)]

            def body_a(dd, accs, loc16=loc16, wbc=wbc, lshc=lshc,
                       wbx=wbx, lshx=lshx, seln=seln, rowk=rowk):
                wrap = (dd + loc16) & (EMBED - 1)
                v = unpack(plsc.load_gather(rows_c, [loc16, wbc + wrap]),
                           lshc)
                up = unpack(plsc.load_gather(rows_x, [loc16, wbx + wrap]),
                            lshx)
                new = [accs[0] + v * up]
                for k in range(KH):
                    un = unpack(plsc.load_gather(
                        rows_n, [rowk[k], seln[k][0] + wrap]), seln[k][1])
                    new.append(accs[k + 1] + v * un)
                return tuple(new)

            accs = lax.fori_loop(
                0, EMBED, body_a,
                tuple(jnp.zeros((L,), jnp.float32) for _ in range(KH + 1)))
            plsc.store_scatter(pos_v, [tb], accs[0])
            for k in range(KH):
                plsc.store_scatter(neg_v, [tb20 + k], accs[k + 1])

            seln2 = [wordsel(plsc.load_gather(
                         raw_n, [jnp.full((L,), KH + k, jnp.int32), tb]))
                     for k in range(KH)]
            rowk2 = [rowb + KH + k for k in range(KH)]

            def body_b(dd, accs, loc16=loc16, wbc=wbc, lshc=lshc,
                       seln2=seln2, rowk2=rowk2):
                wrap = (dd + loc16) & (EMBED - 1)
                v = unpack(plsc.load_gather(rows_c, [loc16, wbc + wrap]),
                           lshc)
                new = []
                for k in range(KH):
                    un = unpack(plsc.load_gather(
                        rows_n, [rowk2[k], seln2[k][0] + wrap]),
                        seln2[k][1])
                    new.append(accs[k] + v * un)
                return tuple(new)

            accs = lax.fori_loop(
                0, EMBED, body_b,
                tuple(jnp.zeros((L,), jnp.float32) for _ in range(KH)))
            for k in range(KH):
                plsc.store_scatter(neg_v, [tb20 + KH + k], accs[k])

        issue(0, rows_cA, rows_xA, rows_nA, semA)

        def pair_body(i, _):
            sa = 2 * i
            issue(sa + 1, rows_cB, rows_xB, rows_nB, semB)
            drain(rows_cA, rows_xA, rows_nA, semA)
            compute(sa, rows_cA, rows_xA, rows_nA)

            @pl.when(i < NSTEPS // 2 - 1)
            def _():
                issue(sa + 2, rows_cA, rows_xA, rows_nA, semA)

            drain(rows_cB, rows_xB, rows_nB, semB)
            compute(sa + 1, rows_cB, rows_xB, rows_nB)
            return 0

        lax.fori_loop(0, NSTEPS // 2, pair_body, 0)
        pltpu.sync_copy(pos_v, pos_h.at[pl.ds(base, B_PER_W)])
        pltpu.sync_copy(neg_v, neg_out_h.at[pl.ds(base * NUM_NEG, NEG_PER_W)])

    return scores_kernel(center, context, neg_flat, wc2, wx2)


def _iota16():
    return lax.iota(jnp.int32, L)


def _loss_kernel(pos_ref, neg_ref, out_ref):
    def log_sigmoid(x):
        return jnp.minimum(x, 0.0) - jnp.log1p(jnp.exp(-jnp.abs(x)))

    total = (jnp.sum(log_sigmoid(pos_ref[...]))
             + jnp.sum(log_sigmoid(-neg_ref[...])))
    out_ref[0, 0] = -total / BATCH


def kernel(center, context, negatives, W_center, W_context):
    center = center.astype(jnp.int32)
    context = context.astype(jnp.int32)
    neg_t = negatives.astype(jnp.int32).T
    wc2, wx2 = _transpose_pack(W_center.T, W_context.T)
    pos, neg = _sc_scores(center, context, neg_t, wc2, wx2)
    loss = pl.pallas_call(
        _loss_kernel,
        out_shape=jax.ShapeDtypeStruct((1, 1), jnp.float32),
        in_specs=[
            pl.BlockSpec(memory_space=pltpu.VMEM),
            pl.BlockSpec(memory_space=pltpu.VMEM),
        ],
        out_specs=pl.BlockSpec(memory_space=pltpu.SMEM),
    )(pos.reshape(BATCH // 128, 128), neg.reshape(BATCH * NUM_NEG // 128, 128))
    return loss[0, 0]


# speedup vs baseline: 4.1769x; 1.1701x over previous
import functools

import jax
import jax.numpy as jnp
from jax import lax
from jax.experimental import pallas as pl
from jax.experimental.pallas import tpu as pltpu
from jax.experimental.pallas import tpu_sc as plsc

VOCAB = 1000000
EMBED = 64
BATCH = 16384
NUM_NEG = 20

NC, NS, L = 2, 16, 16
NW = NC * NS
B_PER_W = BATCH // NW
PHYS = 2 * EMBED
CHUNK = 16
NSTEPS = B_PER_W // CHUNK
NEG_ROWS = CHUNK * NUM_NEG
NSEG = (128, 128, 64)
NEG_PER_W = B_PER_W * NUM_NEG

TBLK = 2560
NTBLK = 98
QUART = NTBLK * TBLK
NCOLB = (VOCAB + TBLK - 1) // TBLK - 1


def _transpose_pack(wt_c, wt_x):
    def body(c0, c1, c2, c3, x0, x1, x2, x3, oc_ref, ox_ref):
        eye = (lax.broadcasted_iota(jnp.int32, (EMBED, EMBED), 0)
               == lax.broadcasted_iota(jnp.int32, (EMBED, EMBED), 1)
               ).astype(jnp.bfloat16)

        def tr(ref):
            return lax.dot_general(ref[...].astype(jnp.bfloat16), eye,
                                   (((0,), (0,)), ((), ())),
                                   preferred_element_type=jnp.float32)

        def pack(lo, hi):
            lo_bits = lax.bitcast_convert_type(lo, jnp.uint32)
            hi_bits = lax.bitcast_convert_type(hi, jnp.uint32)
            word = (lo_bits >> 16) | (hi_bits & jnp.uint32(0xFFFF0000))
            return lax.bitcast_convert_type(word, jnp.int32)

        def packed(r0, r1, r2, r3):
            return jnp.concatenate(
                [pack(tr(r0), tr(r2)), pack(tr(r1), tr(r3))], axis=1)

        oc_ref[...] = packed(c0, c1, c2, c3)
        ox_ref[...] = packed(x0, x1, x2, x3)

    def qmap(qs):
        if qs == 3 * NTBLK:
            return lambda b: (0, jnp.minimum(qs + b, NCOLB))
        return lambda b: (0, qs + b)

    in_spec = [pl.BlockSpec((EMBED, TBLK), qmap(q * NTBLK))
               for q in range(4)] * 2
    out_spec = pl.BlockSpec((TBLK, PHYS), lambda b: (b, 0))
    return pl.pallas_call(
        body,
        grid=(NTBLK,),
        in_specs=in_spec,
        out_specs=[out_spec, out_spec],
        out_shape=[jax.ShapeDtypeStruct((QUART, PHYS), jnp.int32)] * 2,
    )(wt_c, wt_c, wt_c, wt_c, wt_x, wt_x, wt_x, wt_x)


def _sc_scores(center, context, neg_flat, wc2, wx2):
    mesh = plsc.VectorSubcoreMesh(core_axis_name="c", subcore_axis_name="s")

    @functools.partial(
        pl.kernel,
        out_type=(
            jax.ShapeDtypeStruct((BATCH,), jnp.float32),
            jax.ShapeDtypeStruct((BATCH * NUM_NEG,), jnp.float32),
        ),
        mesh=mesh,
        scratch_types=[
            pltpu.VMEM((B_PER_W,), jnp.int32),
            pltpu.VMEM((B_PER_W,), jnp.int32),
            pltpu.VMEM((NUM_NEG, B_PER_W), jnp.int32),
            pltpu.VMEM((B_PER_W,), jnp.int32),
            pltpu.VMEM((B_PER_W,), jnp.int32),
            pltpu.VMEM((NEG_PER_W,), jnp.int32),
            pltpu.VMEM((CHUNK, PHYS), jnp.int32),
            pltpu.VMEM((CHUNK, PHYS), jnp.int32),
            pltpu.VMEM((NEG_ROWS, PHYS), jnp.int32),
            pltpu.VMEM((CHUNK, PHYS), jnp.int32),
            pltpu.VMEM((CHUNK, PHYS), jnp.int32),
            pltpu.VMEM((NEG_ROWS, PHYS), jnp.int32),
            pltpu.VMEM((B_PER_W,), jnp.float32),
            pltpu.VMEM((NEG_PER_W,), jnp.float32),
            pltpu.SemaphoreType.DMA,
            pltpu.SemaphoreType.DMA,
        ],
        compiler_params=pltpu.CompilerParams(
            needs_layout_passes=False, use_tc_tiling_on_sc=True),
    )
    def scores_kernel(center_h, context_h, neg_h, wc_h, wx_h,
                      pos_h, neg_out_h,
                      raw_c, raw_x, raw_n, row_c, row_x, row_n,
                      rows_cA, rows_xA, rows_nA, rows_cB, rows_xB, rows_nB,
                      pos_v, neg_v, semA, semB):
        wid = lax.axis_index("s") * NC + lax.axis_index("c")
        base = wid * B_PER_W

        pltpu.sync_copy(center_h.at[pl.ds(base, B_PER_W)], raw_c)
        pltpu.sync_copy(context_h.at[pl.ds(base, B_PER_W)], raw_x)
        pltpu.sync_copy(neg_h.at[:, pl.ds(base, B_PER_W)], raw_n)

        def quad_of(x):
            return ((x >= QUART).astype(jnp.int32)
                    + (x >= 2 * QUART).astype(jnp.int32)
                    + (x >= 3 * QUART).astype(jnp.int32))

        def shift_body(i, _, src, dst):
            v16 = i * L + _iota16()
            x = plsc.load_gather(src, [v16])
            plsc.store_scatter(dst, [v16], x - quad_of(x) * QUART)
            return 0

        lax.fori_loop(0, B_PER_W // L,
                      functools.partial(shift_body, src=raw_c, dst=row_c), 0)
        lax.fori_loop(0, B_PER_W // L,
                      functools.partial(shift_body, src=raw_x, dst=row_x), 0)

        def neg_shift_body(j, _, k):
            v16 = j * L + _iota16()
            x = plsc.load_gather(raw_n, [jnp.full((L,), k, jnp.int32), v16])
            plsc.store_scatter(row_n, [v16 * NUM_NEG + k],
                               x - quad_of(x) * QUART)
            return 0

        for k in range(NUM_NEG):
            lax.fori_loop(0, B_PER_W // L,
                          functools.partial(neg_shift_body, k=k), 0)

        def wordsel(raw_vec):
            quad = quad_of(raw_vec)
            wb = (quad & 1) << 6
            lsh = jnp.where(quad >= 2, 0, 16).astype(jnp.uint32)
            return wb, lsh

        def unpack(word, lsh):
            u = plsc.bitcast(word, jnp.uint32)
            bits = (u << lsh) & jnp.uint32(0xFFFF0000)
            return plsc.bitcast(bits, jnp.float32)

        def issue(step, rows_c, rows_x, rows_n, sem):
            cb = step * CHUNK
            nb = step * NEG_ROWS
            pltpu.async_copy(wc_h.at[row_c.at[pl.ds(cb, CHUNK)]],
                             rows_c, sem)
            pltpu.async_copy(wx_h.at[row_x.at[pl.ds(cb, CHUNK)]],
                             rows_x, sem)
            off = 0
            for seg in NSEG:
                pltpu.async_copy(wx_h.at[row_n.at[pl.ds(nb + off, seg)]],
                                 rows_n.at[pl.ds(off, seg)], sem)
                off += seg

        def drain(rows_c, rows_x, rows_n, sem):
            pltpu.make_async_copy(wc_h.at[pl.ds(0, CHUNK)], rows_c,
                                  sem).wait()
            pltpu.make_async_copy(wc_h.at[pl.ds(0, CHUNK)], rows_x,
                                  sem).wait()
            off = 0
            for seg in NSEG:
                pltpu.make_async_copy(wc_h.at[pl.ds(0, seg)],
                                      rows_n.at[pl.ds(off, seg)],
                                      sem).wait()
                off += seg

        def compute(step, rows_c, rows_x, rows_n):
          for g in range(CHUNK // L):
            loc16 = _iota16() + g * L
            tb = loc16 + step * CHUNK
            tb20 = tb * NUM_NEG
            rowb = loc16 * NUM_NEG
            wbc, lshc = wordsel(plsc.load_gather(raw_c, [tb]))
            KH = NUM_NEG // 2

            wbx, lshx = wordsel(plsc.load_gather(raw_x, [tb]))
            seln = [wordsel(plsc.load_gather(
                        raw_n, [jnp.full((L,), k, jnp.int32), tb]))
                    for k in range(KH)]
            rowk = [rowb + k for k in range(KH)]

            def body_a(dd, accs, loc16=loc16, wbc=wbc, lshc=lshc,
                       wbx=wbx, lshx=lshx, seln=seln, rowk=rowk):
                wrap = (dd + loc16) & (EMBED - 1)
                v = unpack(plsc.load_gather(rows_c, [loc16, wbc + wrap]),
                           lshc)
                up = unpack(plsc.load_gather(rows_x, [loc16, wbx + wrap]),
                            lshx)
                new = [accs[0] + v * up]
                for k in range(KH):
                    un = unpack(plsc.load_gather(
                        rows_n, [rowk[k], seln[k][0] + wrap]), seln[k][1])
                    new.append(accs[k + 1] + v * un)
                return tuple(new)

            accs = lax.fori_loop(
                0, EMBED, body_a,
                tuple(jnp.zeros((L,), jnp.float32) for _ in range(KH + 1)))
            plsc.store_scatter(pos_v, [tb], accs[0])
            for k in range(KH):
                plsc.store_scatter(neg_v, [tb20 + k], accs[k + 1])

            seln2 = [wordsel(plsc.load_gather(
                         raw_n, [jnp.full((L,), KH + k, jnp.int32), tb]))
                     for k in range(KH)]
            rowk2 = [rowb + KH + k for k in range(KH)]

            def body_b(dd, accs, loc16=loc16, wbc=wbc, lshc=lshc,
                       seln2=seln2, rowk2=rowk2):
                wrap = (dd + loc16) & (EMBED - 1)
                v = unpack(plsc.load_gather(rows_c, [loc16, wbc + wrap]),
                           lshc)
                new = []
                for k in range(KH):
                    un = unpack(plsc.load_gather(
                        rows_n, [rowk2[k], seln2[k][0] + wrap]),
                        seln2[k][1])
                    new.append(accs[k] + v * un)
                return tuple(new)

            accs = lax.fori_loop(
                0, EMBED, body_b,
                tuple(jnp.zeros((L,), jnp.float32) for _ in range(KH)))
            for k in range(KH):
                plsc.store_scatter(neg_v, [tb20 + KH + k], accs[k])

        issue(0, rows_cA, rows_xA, rows_nA, semA)

        def pair_body(i, _):
            sa = 2 * i
            issue(sa + 1, rows_cB, rows_xB, rows_nB, semB)
            drain(rows_cA, rows_xA, rows_nA, semA)
            compute(sa, rows_cA, rows_xA, rows_nA)

            @pl.when(i < NSTEPS // 2 - 1)
            def _():
                issue(sa + 2, rows_cA, rows_xA, rows_nA, semA)

            drain(rows_cB, rows_xB, rows_nB, semB)
            compute(sa + 1, rows_cB, rows_xB, rows_nB)
            return 0

        lax.fori_loop(0, NSTEPS // 2, pair_body, 0)
        pltpu.sync_copy(pos_v, pos_h.at[pl.ds(base, B_PER_W)])
        pltpu.sync_copy(neg_v, neg_out_h.at[pl.ds(base * NUM_NEG, NEG_PER_W)])

    return scores_kernel(center, context, neg_flat, wc2, wx2)


def _iota16():
    return lax.iota(jnp.int32, L)


def _loss_kernel(pos_ref, neg_ref, out_ref):
    def log_sigmoid(x):
        return jnp.minimum(x, 0.0) - jnp.log1p(jnp.exp(-jnp.abs(x)))

    total = (jnp.sum(log_sigmoid(pos_ref[...]))
             + jnp.sum(log_sigmoid(-neg_ref[...])))
    out_ref[0, 0] = -total / BATCH


def kernel(center, context, negatives, W_center, W_context):
    center = center.astype(jnp.int32)
    context = context.astype(jnp.int32)
    neg_t = negatives.astype(jnp.int32).T
    wc2, wx2 = _transpose_pack(W_center.T, W_context.T)
    pos, neg = _sc_scores(center, context, neg_t, wc2, wx2)
    loss = pl.pallas_call(
        _loss_kernel,
        out_shape=jax.ShapeDtypeStruct((1, 1), jnp.float32),
        in_specs=[
            pl.BlockSpec(memory_space=pltpu.VMEM),
            pl.BlockSpec(memory_space=pltpu.VMEM),
        ],
        out_specs=pl.BlockSpec(memory_space=pltpu.SMEM),
    )(pos.reshape(BATCH // 128, 128), neg.reshape(BATCH * NUM_NEG // 128, 128))
    return loss[0, 0]


# speedup vs baseline: 4.5304x; 1.0846x over previous
import functools

import jax
import jax.numpy as jnp
from jax import lax
from jax.experimental import pallas as pl
from jax.experimental.pallas import tpu as pltpu
from jax.experimental.pallas import tpu_sc as plsc

VOCAB = 1000000
EMBED = 64
BATCH = 16384
NUM_NEG = 20

NC, NS, L = 2, 16, 16
NW = NC * NS
B_PER_W = BATCH // NW
PHYS = 2 * EMBED
CHUNK = 16
NSTEPS = B_PER_W // CHUNK
NEG_ROWS = CHUNK * NUM_NEG
NSEG = (128, 128, 64)
NEG_PER_W = B_PER_W * NUM_NEG

TBLK = 5120
NTBLK = 49
QUART = NTBLK * TBLK
NCOLB = (VOCAB + TBLK - 1) // TBLK - 1


def _transpose_pack(wt_c, wt_x):
    def body(c0, c1, c2, c3, x0, x1, x2, x3, oc_ref, ox_ref):
        eye = (lax.broadcasted_iota(jnp.int32, (EMBED, EMBED), 0)
               == lax.broadcasted_iota(jnp.int32, (EMBED, EMBED), 1)
               ).astype(jnp.bfloat16)

        def tr(ref):
            return lax.dot_general(ref[...].astype(jnp.bfloat16), eye,
                                   (((0,), (0,)), ((), ())),
                                   preferred_element_type=jnp.float32)

        def pack(lo, hi):
            lo_bits = lax.bitcast_convert_type(lo, jnp.uint32)
            hi_bits = lax.bitcast_convert_type(hi, jnp.uint32)
            word = (lo_bits >> 16) | (hi_bits & jnp.uint32(0xFFFF0000))
            return lax.bitcast_convert_type(word, jnp.int32)

        def packed(r0, r1, r2, r3):
            return jnp.concatenate(
                [pack(tr(r0), tr(r2)), pack(tr(r1), tr(r3))], axis=1)

        oc_ref[...] = packed(c0, c1, c2, c3)
        ox_ref[...] = packed(x0, x1, x2, x3)

    def qmap(qs):
        if qs == 3 * NTBLK:
            return lambda b: (0, jnp.minimum(qs + b, NCOLB))
        return lambda b: (0, qs + b)

    in_spec = [pl.BlockSpec((EMBED, TBLK), qmap(q * NTBLK))
               for q in range(4)] * 2
    out_spec = pl.BlockSpec((TBLK, PHYS), lambda b: (b, 0))
    return pl.pallas_call(
        body,
        grid=(NTBLK,),
        in_specs=in_spec,
        out_specs=[out_spec, out_spec],
        out_shape=[jax.ShapeDtypeStruct((QUART, PHYS), jnp.int32)] * 2,
    )(wt_c, wt_c, wt_c, wt_c, wt_x, wt_x, wt_x, wt_x)


def _sc_scores(center, context, neg_flat, wc2, wx2):
    mesh = plsc.VectorSubcoreMesh(core_axis_name="c", subcore_axis_name="s")

    @functools.partial(
        pl.kernel,
        out_type=(
            jax.ShapeDtypeStruct((BATCH,), jnp.float32),
            jax.ShapeDtypeStruct((BATCH * NUM_NEG,), jnp.float32),
        ),
        mesh=mesh,
        scratch_types=[
            pltpu.VMEM((B_PER_W,), jnp.int32),
            pltpu.VMEM((B_PER_W,), jnp.int32),
            pltpu.VMEM((NUM_NEG, B_PER_W), jnp.int32),
            pltpu.VMEM((B_PER_W,), jnp.int32),
            pltpu.VMEM((B_PER_W,), jnp.int32),
            pltpu.VMEM((NEG_PER_W,), jnp.int32),
            pltpu.VMEM((CHUNK, PHYS), jnp.int32),
            pltpu.VMEM((CHUNK, PHYS), jnp.int32),
            pltpu.VMEM((NEG_ROWS, PHYS), jnp.int32),
            pltpu.VMEM((CHUNK, PHYS), jnp.int32),
            pltpu.VMEM((CHUNK, PHYS), jnp.int32),
            pltpu.VMEM((NEG_ROWS, PHYS), jnp.int32),
            pltpu.VMEM((B_PER_W,), jnp.float32),
            pltpu.VMEM((NEG_PER_W,), jnp.float32),
            pltpu.SemaphoreType.DMA,
            pltpu.SemaphoreType.DMA,
        ],
        compiler_params=pltpu.CompilerParams(
            needs_layout_passes=False, use_tc_tiling_on_sc=True),
    )
    def scores_kernel(center_h, context_h, neg_h, wc_h, wx_h,
                      pos_h, neg_out_h,
                      raw_c, raw_x, raw_n, row_c, row_x, row_n,
                      rows_cA, rows_xA, rows_nA, rows_cB, rows_xB, rows_nB,
                      pos_v, neg_v, semA, semB):
        wid = lax.axis_index("s") * NC + lax.axis_index("c")
        base = wid * B_PER_W

        pltpu.sync_copy(center_h.at[pl.ds(base, B_PER_W)], raw_c)
        pltpu.sync_copy(context_h.at[pl.ds(base, B_PER_W)], raw_x)
        pltpu.sync_copy(neg_h.at[:, pl.ds(base, B_PER_W)], raw_n)

        def quad_of(x):
            return ((x >= QUART).astype(jnp.int32)
                    + (x >= 2 * QUART).astype(jnp.int32)
                    + (x >= 3 * QUART).astype(jnp.int32))

        def shift_body(i, _, src, dst):
            v16 = i * L + _iota16()
            x = plsc.load_gather(src, [v16])
            plsc.store_scatter(dst, [v16], x - quad_of(x) * QUART)
            return 0

        lax.fori_loop(0, B_PER_W // L,
                      functools.partial(shift_body, src=raw_c, dst=row_c), 0)
        lax.fori_loop(0, B_PER_W // L,
                      functools.partial(shift_body, src=raw_x, dst=row_x), 0)

        def neg_shift_body(j, _, k):
            v16 = j * L + _iota16()
            x = plsc.load_gather(raw_n, [jnp.full((L,), k, jnp.int32), v16])
            plsc.store_scatter(row_n, [v16 * NUM_NEG + k],
                               x - quad_of(x) * QUART)
            return 0

        for k in range(NUM_NEG):
            lax.fori_loop(0, B_PER_W // L,
                          functools.partial(neg_shift_body, k=k), 0)

        def wordsel(raw_vec):
            quad = quad_of(raw_vec)
            wb = (quad & 1) << 6
            lsh = jnp.where(quad >= 2, 0, 16).astype(jnp.uint32)
            return wb, lsh

        def unpack(word, lsh):
            u = plsc.bitcast(word, jnp.uint32)
            bits = (u << lsh) & jnp.uint32(0xFFFF0000)
            return plsc.bitcast(bits, jnp.float32)

        def issue(step, rows_c, rows_x, rows_n, sem):
            cb = step * CHUNK
            nb = step * NEG_ROWS
            pltpu.async_copy(wc_h.at[row_c.at[pl.ds(cb, CHUNK)]],
                             rows_c, sem)
            pltpu.async_copy(wx_h.at[row_x.at[pl.ds(cb, CHUNK)]],
                             rows_x, sem)
            off = 0
            for seg in NSEG:
                pltpu.async_copy(wx_h.at[row_n.at[pl.ds(nb + off, seg)]],
                                 rows_n.at[pl.ds(off, seg)], sem)
                off += seg

        def drain(rows_c, rows_x, rows_n, sem):
            pltpu.make_async_copy(wc_h.at[pl.ds(0, CHUNK)], rows_c,
                                  sem).wait()
            pltpu.make_async_copy(wc_h.at[pl.ds(0, CHUNK)], rows_x,
                                  sem).wait()
            off = 0
            for seg in NSEG:
                pltpu.make_async_copy(wc_h.at[pl.ds(0, seg)],
                                      rows_n.at[pl.ds(off, seg)],
                                      sem).wait()
                off += seg

        def compute(step, rows_c, rows_x, rows_n):
          for g in range(CHUNK // L):
            loc16 = _iota16() + g * L
            tb = loc16 + step * CHUNK
            tb20 = tb * NUM_NEG
            rowb = loc16 * NUM_NEG
            wbc, lshc = wordsel(plsc.load_gather(raw_c, [tb]))
            KH = NUM_NEG // 2

            wbx, lshx = wordsel(plsc.load_gather(raw_x, [tb]))
            seln = [wordsel(plsc.load_gather(
                        raw_n, [jnp.full((L,), k, jnp.int32), tb]))
                    for k in range(KH)]
            rowk = [rowb + k for k in range(KH)]

            def body_a(dd, accs, loc16=loc16, wbc=wbc, lshc=lshc,
                       wbx=wbx, lshx=lshx, seln=seln, rowk=rowk):
                wrap = (dd + loc16) & (EMBED - 1)
                v = unpack(plsc.load_gather(rows_c, [loc16, wbc + wrap]),
                           lshc)
                up = unpack(plsc.load_gather(rows_x, [loc16, wbx + wrap]),
                            lshx)
                new = [accs[0] + v * up]
                for k in range(KH):
                    un = unpack(plsc.load_gather(
                        rows_n, [rowk[k], seln[k][0] + wrap]), seln[k][1])
                    new.append(accs[k + 1] + v * un)
                return tuple(new)

            accs = lax.fori_loop(
                0, EMBED, body_a,
                tuple(jnp.zeros((L,), jnp.float32) for _ in range(KH + 1)))
            plsc.store_scatter(pos_v, [tb], accs[0])
            for k in range(KH):
                plsc.store_scatter(neg_v, [tb20 + k], accs[k + 1])

            seln2 = [wordsel(plsc.load_gather(
                         raw_n, [jnp.full((L,), KH + k, jnp.int32), tb]))
                     for k in range(KH)]
            rowk2 = [rowb + KH + k for k in range(KH)]

            def body_b(dd, accs, loc16=loc16, wbc=wbc, lshc=lshc,
                       seln2=seln2, rowk2=rowk2):
                wrap = (dd + loc16) & (EMBED - 1)
                v = unpack(plsc.load_gather(rows_c, [loc16, wbc + wrap]),
                           lshc)
                new = []
                for k in range(KH):
                    un = unpack(plsc.load_gather(
                        rows_n, [rowk2[k], seln2[k][0] + wrap]),
                        seln2[k][1])
                    new.append(accs[k] + v * un)
                return tuple(new)

            accs = lax.fori_loop(
                0, EMBED, body_b,
                tuple(jnp.zeros((L,), jnp.float32) for _ in range(KH)))
            for k in range(KH):
                plsc.store_scatter(neg_v, [tb20 + KH + k], accs[k])

        issue(0, rows_cA, rows_xA, rows_nA, semA)

        def pair_body(i, _):
            sa = 2 * i
            issue(sa + 1, rows_cB, rows_xB, rows_nB, semB)
            drain(rows_cA, rows_xA, rows_nA, semA)
            compute(sa, rows_cA, rows_xA, rows_nA)

            @pl.when(i < NSTEPS // 2 - 1)
            def _():
                issue(sa + 2, rows_cA, rows_xA, rows_nA, semA)

            drain(rows_cB, rows_xB, rows_nB, semB)
            compute(sa + 1, rows_cB, rows_xB, rows_nB)
            return 0

        lax.fori_loop(0, NSTEPS // 2, pair_body, 0)
        pltpu.sync_copy(pos_v, pos_h.at[pl.ds(base, B_PER_W)])
        pltpu.sync_copy(neg_v, neg_out_h.at[pl.ds(base * NUM_NEG, NEG_PER_W)])

    return scores_kernel(center, context, neg_flat, wc2, wx2)


def _iota16():
    return lax.iota(jnp.int32, L)


def _loss_kernel(pos_ref, neg_ref, out_ref):
    def log_sigmoid(x):
        return jnp.minimum(x, 0.0) - jnp.log1p(jnp.exp(-jnp.abs(x)))

    total = (jnp.sum(log_sigmoid(pos_ref[...]))
             + jnp.sum(log_sigmoid(-neg_ref[...])))
    out_ref[0, 0] = -total / BATCH


def kernel(center, context, negatives, W_center, W_context):
    center = center.astype(jnp.int32)
    context = context.astype(jnp.int32)
    neg_t = negatives.astype(jnp.int32).T
    wc2, wx2 = _transpose_pack(W_center.T, W_context.T)
    pos, neg = _sc_scores(center, context, neg_t, wc2, wx2)
    loss = pl.pallas_call(
        _loss_kernel,
        out_shape=jax.ShapeDtypeStruct((1, 1), jnp.float32),
        in_specs=[
            pl.BlockSpec(memory_space=pltpu.VMEM),
            pl.BlockSpec(memory_space=pltpu.VMEM),
        ],
        out_specs=pl.BlockSpec(memory_space=pltpu.SMEM),
    )(pos.reshape(BATCH // 128, 128), neg.reshape(BATCH * NUM_NEG // 128, 128))
    return loss[0, 0]


# speedup vs baseline: 4.6407x; 1.0243x over previous
import functools

import jax
import jax.numpy as jnp
from jax import lax
from jax.experimental import pallas as pl
from jax.experimental.pallas import tpu as pltpu
from jax.experimental.pallas import tpu_sc as plsc

VOCAB = 1000000
EMBED = 64
BATCH = 16384
NUM_NEG = 20

NC, NS, L = 2, 16, 16
NW = NC * NS
B_PER_W = BATCH // NW
PHYS = 2 * EMBED
CHUNK = 16
NSTEPS = B_PER_W // CHUNK
NEG_ROWS = CHUNK * NUM_NEG
NSEG = (128, 128, 64)
NEG_PER_W = B_PER_W * NUM_NEG

TBLK = 7168
NTBLK = 35
QUART = NTBLK * TBLK
NCOLB = (VOCAB + TBLK - 1) // TBLK - 1


def _transpose_pack(wt_c, wt_x):
    def body(c0, c1, c2, c3, x0, x1, x2, x3, oc_ref, ox_ref):
        eye = (lax.broadcasted_iota(jnp.int32, (EMBED, EMBED), 0)
               == lax.broadcasted_iota(jnp.int32, (EMBED, EMBED), 1)
               ).astype(jnp.bfloat16)

        def tr(ref):
            return lax.dot_general(ref[...].astype(jnp.bfloat16), eye,
                                   (((0,), (0,)), ((), ())),
                                   preferred_element_type=jnp.float32)

        def pack(lo, hi):
            lo_bits = lax.bitcast_convert_type(lo, jnp.uint32)
            hi_bits = lax.bitcast_convert_type(hi, jnp.uint32)
            word = (lo_bits >> 16) | (hi_bits & jnp.uint32(0xFFFF0000))
            return lax.bitcast_convert_type(word, jnp.int32)

        def packed(r0, r1, r2, r3):
            return jnp.concatenate(
                [pack(tr(r0), tr(r2)), pack(tr(r1), tr(r3))], axis=1)

        oc_ref[...] = packed(c0, c1, c2, c3)
        ox_ref[...] = packed(x0, x1, x2, x3)

    def qmap(qs):
        if qs == 3 * NTBLK:
            return lambda b: (0, jnp.minimum(qs + b, NCOLB))
        return lambda b: (0, qs + b)

    in_spec = [pl.BlockSpec((EMBED, TBLK), qmap(q * NTBLK))
               for q in range(4)] * 2
    out_spec = pl.BlockSpec((TBLK, PHYS), lambda b: (b, 0))
    return pl.pallas_call(
        body,
        grid=(NTBLK,),
        in_specs=in_spec,
        out_specs=[out_spec, out_spec],
        out_shape=[jax.ShapeDtypeStruct((QUART, PHYS), jnp.int32)] * 2,
    )(wt_c, wt_c, wt_c, wt_c, wt_x, wt_x, wt_x, wt_x)


def _sc_scores(center, context, neg_flat, wc2, wx2):
    mesh = plsc.VectorSubcoreMesh(core_axis_name="c", subcore_axis_name="s")

    @functools.partial(
        pl.kernel,
        out_type=(
            jax.ShapeDtypeStruct((BATCH,), jnp.float32),
            jax.ShapeDtypeStruct((BATCH * NUM_NEG,), jnp.float32),
        ),
        mesh=mesh,
        scratch_types=[
            pltpu.VMEM((B_PER_W,), jnp.int32),
            pltpu.VMEM((B_PER_W,), jnp.int32),
            pltpu.VMEM((NUM_NEG, B_PER_W), jnp.int32),
            pltpu.VMEM((B_PER_W,), jnp.int32),
            pltpu.VMEM((B_PER_W,), jnp.int32),
            pltpu.VMEM((NEG_PER_W,), jnp.int32),
            pltpu.VMEM((CHUNK, PHYS), jnp.int32),
            pltpu.VMEM((CHUNK, PHYS), jnp.int32),
            pltpu.VMEM((NEG_ROWS, PHYS), jnp.int32),
            pltpu.VMEM((CHUNK, PHYS), jnp.int32),
            pltpu.VMEM((CHUNK, PHYS), jnp.int32),
            pltpu.VMEM((NEG_ROWS, PHYS), jnp.int32),
            pltpu.VMEM((B_PER_W,), jnp.float32),
            pltpu.VMEM((NEG_PER_W,), jnp.float32),
            pltpu.SemaphoreType.DMA,
            pltpu.SemaphoreType.DMA,
        ],
        compiler_params=pltpu.CompilerParams(
            needs_layout_passes=False, use_tc_tiling_on_sc=True),
    )
    def scores_kernel(center_h, context_h, neg_h, wc_h, wx_h,
                      pos_h, neg_out_h,
                      raw_c, raw_x, raw_n, row_c, row_x, row_n,
                      rows_cA, rows_xA, rows_nA, rows_cB, rows_xB, rows_nB,
                      pos_v, neg_v, semA, semB):
        wid = lax.axis_index("s") * NC + lax.axis_index("c")
        base = wid * B_PER_W

        pltpu.sync_copy(center_h.at[pl.ds(base, B_PER_W)], raw_c)
        pltpu.sync_copy(context_h.at[pl.ds(base, B_PER_W)], raw_x)
        pltpu.sync_copy(neg_h.at[:, pl.ds(base, B_PER_W)], raw_n)

        def quad_of(x):
            return ((x >= QUART).astype(jnp.int32)
                    + (x >= 2 * QUART).astype(jnp.int32)
                    + (x >= 3 * QUART).astype(jnp.int32))

        def shift_body(i, _, src, dst):
            v16 = i * L + _iota16()
            x = plsc.load_gather(src, [v16])
            plsc.store_scatter(dst, [v16], x - quad_of(x) * QUART)
            return 0

        lax.fori_loop(0, B_PER_W // L,
                      functools.partial(shift_body, src=raw_c, dst=row_c), 0)
        lax.fori_loop(0, B_PER_W // L,
                      functools.partial(shift_body, src=raw_x, dst=row_x), 0)

        def neg_shift_body(j, _, k):
            v16 = j * L + _iota16()
            x = plsc.load_gather(raw_n, [jnp.full((L,), k, jnp.int32), v16])
            plsc.store_scatter(row_n, [v16 * NUM_NEG + k],
                               x - quad_of(x) * QUART)
            return 0

        for k in range(NUM_NEG):
            lax.fori_loop(0, B_PER_W // L,
                          functools.partial(neg_shift_body, k=k), 0)

        def wordsel(raw_vec):
            quad = quad_of(raw_vec)
            wb = (quad & 1) << 6
            lsh = jnp.where(quad >= 2, 0, 16).astype(jnp.uint32)
            return wb, lsh

        def unpack(word, lsh):
            u = plsc.bitcast(word, jnp.uint32)
            bits = (u << lsh) & jnp.uint32(0xFFFF0000)
            return plsc.bitcast(bits, jnp.float32)

        def issue(step, rows_c, rows_x, rows_n, sem):
            cb = step * CHUNK
            nb = step * NEG_ROWS
            pltpu.async_copy(wc_h.at[row_c.at[pl.ds(cb, CHUNK)]],
                             rows_c, sem)
            pltpu.async_copy(wx_h.at[row_x.at[pl.ds(cb, CHUNK)]],
                             rows_x, sem)
            off = 0
            for seg in NSEG:
                pltpu.async_copy(wx_h.at[row_n.at[pl.ds(nb + off, seg)]],
                                 rows_n.at[pl.ds(off, seg)], sem)
                off += seg

        def drain(rows_c, rows_x, rows_n, sem):
            pltpu.make_async_copy(wc_h.at[pl.ds(0, CHUNK)], rows_c,
                                  sem).wait()
            pltpu.make_async_copy(wc_h.at[pl.ds(0, CHUNK)], rows_x,
                                  sem).wait()
            off = 0
            for seg in NSEG:
                pltpu.make_async_copy(wc_h.at[pl.ds(0, seg)],
                                      rows_n.at[pl.ds(off, seg)],
                                      sem).wait()
                off += seg

        def compute(step, rows_c, rows_x, rows_n):
          for g in range(CHUNK // L):
            loc16 = _iota16() + g * L
            tb = loc16 + step * CHUNK
            tb20 = tb * NUM_NEG
            rowb = loc16 * NUM_NEG
            wbc, lshc = wordsel(plsc.load_gather(raw_c, [tb]))
            KH = NUM_NEG // 2

            wbx, lshx = wordsel(plsc.load_gather(raw_x, [tb]))
            seln = [wordsel(plsc.load_gather(
                        raw_n, [jnp.full((L,), k, jnp.int32), tb]))
                    for k in range(KH)]
            rowk = [rowb + k for k in range(KH)]

            def body_a(dd, accs, loc16=loc16, wbc=wbc, lshc=lshc,
                       wbx=wbx, lshx=lshx, seln=seln, rowk=rowk):
                wrap = (dd + loc16) & (EMBED - 1)
                v = unpack(plsc.load_gather(rows_c, [loc16, wbc + wrap]),
                           lshc)
                up = unpack(plsc.load_gather(rows_x, [loc16, wbx + wrap]),
                            lshx)
                new = [accs[0] + v * up]
                for k in range(KH):
                    un = unpack(plsc.load_gather(
                        rows_n, [rowk[k], seln[k][0] + wrap]), seln[k][1])
                    new.append(accs[k + 1] + v * un)
                return tuple(new)

            accs = lax.fori_loop(
                0, EMBED, body_a,
                tuple(jnp.zeros((L,), jnp.float32) for _ in range(KH + 1)))
            plsc.store_scatter(pos_v, [tb], accs[0])
            for k in range(KH):
                plsc.store_scatter(neg_v, [tb20 + k], accs[k + 1])

            seln2 = [wordsel(plsc.load_gather(
                         raw_n, [jnp.full((L,), KH + k, jnp.int32), tb]))
                     for k in range(KH)]
            rowk2 = [rowb + KH + k for k in range(KH)]

            def body_b(dd, accs, loc16=loc16, wbc=wbc, lshc=lshc,
                       seln2=seln2, rowk2=rowk2):
                wrap = (dd + loc16) & (EMBED - 1)
                v = unpack(plsc.load_gather(rows_c, [loc16, wbc + wrap]),
                           lshc)
                new = []
                for k in range(KH):
                    un = unpack(plsc.load_gather(
                        rows_n, [rowk2[k], seln2[k][0] + wrap]),
                        seln2[k][1])
                    new.append(accs[k] + v * un)
                return tuple(new)

            accs = lax.fori_loop(
                0, EMBED, body_b,
                tuple(jnp.zeros((L,), jnp.float32) for _ in range(KH)))
            for k in range(KH):
                plsc.store_scatter(neg_v, [tb20 + KH + k], accs[k])

        issue(0, rows_cA, rows_xA, rows_nA, semA)

        def pair_body(i, _):
            sa = 2 * i
            issue(sa + 1, rows_cB, rows_xB, rows_nB, semB)
            drain(rows_cA, rows_xA, rows_nA, semA)
            compute(sa, rows_cA, rows_xA, rows_nA)

            @pl.when(i < NSTEPS // 2 - 1)
            def _():
                issue(sa + 2, rows_cA, rows_xA, rows_nA, semA)

            drain(rows_cB, rows_xB, rows_nB, semB)
            compute(sa + 1, rows_cB, rows_xB, rows_nB)
            return 0

        lax.fori_loop(0, NSTEPS // 2, pair_body, 0)
        pltpu.sync_copy(pos_v, pos_h.at[pl.ds(base, B_PER_W)])
        pltpu.sync_copy(neg_v, neg_out_h.at[pl.ds(base * NUM_NEG, NEG_PER_W)])

    return scores_kernel(center, context, neg_flat, wc2, wx2)


def _iota16():
    return lax.iota(jnp.int32, L)


def _loss_kernel(pos_ref, neg_ref, out_ref):
    def log_sigmoid(x):
        return jnp.minimum(x, 0.0) - jnp.log1p(jnp.exp(-jnp.abs(x)))

    total = (jnp.sum(log_sigmoid(pos_ref[...]))
             + jnp.sum(log_sigmoid(-neg_ref[...])))
    out_ref[0, 0] = -total / BATCH


def kernel(center, context, negatives, W_center, W_context):
    center = center.astype(jnp.int32)
    context = context.astype(jnp.int32)
    neg_t = negatives.astype(jnp.int32).T
    wc2, wx2 = _transpose_pack(W_center.T, W_context.T)
    pos, neg = _sc_scores(center, context, neg_t, wc2, wx2)
    loss = pl.pallas_call(
        _loss_kernel,
        out_shape=jax.ShapeDtypeStruct((1, 1), jnp.float32),
        in_specs=[
            pl.BlockSpec(memory_space=pltpu.VMEM),
            pl.BlockSpec(memory_space=pltpu.VMEM),
        ],
        out_specs=pl.BlockSpec(memory_space=pltpu.SMEM),
    )(pos.reshape(BATCH // 128, 128), neg.reshape(BATCH * NUM_NEG // 128, 128))
    return loss[0, 0]
